# Initial kernel scaffold; baseline (speedup 1.0000x reference)
import jax
import jax.numpy as jnp
from jax.experimental import pallas as pl


def kernel(user_idx, edge_index, x_table, Win0, bin0, Win1, bin1, ln0_g, ln0_b, ln1_g, ln1_b, tm0_W, tm0_b, tm1_W, tm1_b, tmn0_g, tmn0_b, tmn1_g, tmn1_b, Wout, bout):
    raise NotImplementedError("write your pallas kernel here")


# speedup vs baseline: 2.7350x; 2.7350x over previous
import functools

import jax
import jax.numpy as jnp
from jax import lax
from jax.experimental import pallas as pl
from jax.experimental.pallas import tpu as pltpu
from jax.experimental.pallas import tpu_sc as plsc

N = 10000
USERS = 2000
HID = 256
Q = 64
NQ = 4
CH = 64
OUTD = 128
E = 160000

NC = 2
NS = 16
EC = 128
NCHUNK = 80
E_PAD = NS * NCHUNK * EC
N_PAD = 10112
RPT = N_PAD // NS

BLK = 1000


def _ln(x, g, b, eps=1e-5):
    mu = jnp.mean(x, axis=-1, keepdims=True)
    var = jnp.mean((x - mu) ** 2, axis=-1, keepdims=True)
    return (x - mu) / jnp.sqrt(var + eps) * g + b


def _mlp_body(x_ref, w0_ref, b0_ref, w1_ref, b1_ref, g0_ref, bb0_ref,
              g1_ref, bb1_ref, *o_refs):
    x = x_ref[...]
    h = jnp.maximum(jnp.dot(x, w0_ref[...], preferred_element_type=jnp.float32)
                    + b0_ref[...], 0.0)
    h = _ln(h, g0_ref[...], bb0_ref[...])
    h = jnp.maximum(jnp.dot(h, w1_ref[...], preferred_element_type=jnp.float32)
                    + b1_ref[...], 0.0)
    h = _ln(h, g1_ref[...], bb1_ref[...])
    for i in range(NQ):
        o_refs[i][...] = h[:, i * Q:(i + 1) * Q]


def _mlp(x, w0, b0, w1, b1, g0, bb0, g1, bb1):
    grid = N // BLK
    full = lambda r, c: pl.BlockSpec((r, c), lambda i: (0, 0))
    return pl.pallas_call(
        _mlp_body,
        grid=(grid,),
        in_specs=[
            pl.BlockSpec((BLK, HID), lambda i: (i, 0)),
            full(HID, HID), full(1, HID), full(HID, HID), full(1, HID),
            full(1, HID), full(1, HID), full(1, HID), full(1, HID),
        ],
        out_specs=[pl.BlockSpec((BLK, Q), lambda i: (i, 0))] * NQ,
        out_shape=[jax.ShapeDtypeStruct((N, Q), jnp.float32)] * NQ,
    )(x, w0, b0.reshape(1, HID), w1, b1.reshape(1, HID),
      g0.reshape(1, HID), bb0.reshape(1, HID), g1.reshape(1, HID),
      bb1.reshape(1, HID))


def _segsum_pallas(xqs, src_t, dst_t, zerosq, zeros16, ones16, with_counts):
    out_type = [jax.ShapeDtypeStruct((N_PAD, Q), jnp.float32)] * NQ
    if with_counts:
        out_type = out_type + [jax.ShapeDtypeStruct((N_PAD, 16), jnp.float32)]
    scratch = [
        pltpu.VMEM((NCHUNK, EC), jnp.int32),
        pltpu.VMEM((NCHUNK, EC), jnp.int32),
        pltpu.VMEM((RPT, Q), jnp.float32),
        pltpu.VMEM((RPT, 16), jnp.float32),
        pltpu.VMEM((EC, 16), jnp.float32),
        pltpu.VMEM_SHARED((N_PAD, Q), jnp.float32),
        pltpu.VMEM_SHARED((N_PAD, 16), jnp.float32),
        pltpu.SemaphoreType.DMA,
    ]
    mesh = plsc.VectorSubcoreMesh(core_axis_name="c", subcore_axis_name="s")

    @functools.partial(pl.kernel, out_type=out_type, mesh=mesh,
                       scratch_types=scratch,
                       compiler_params=pltpu.CompilerParams(
                           use_tc_tiling_on_sc=False))
    def k(xq0, xq1, xq2, xq3, src_hbm, dst_hbm, zq_hbm, z16_hbm, ones_hbm,
          oq0, oq1, oq2, oq3, *rest):
        if with_counts:
            ocnt_hbm = rest[0]
            rest = rest[1:]
        src_v, dst_v, rows_v, cbuf_v, ones_v, acc_sh, cnt_sh, sem = rest
        xq = (xq0, xq1, xq2, xq3)
        oq = (oq0, oq1, oq2, oq3)
        c = lax.axis_index("c")
        s = lax.axis_index("s")
        r0 = s * RPT

        pltpu.sync_copy(src_hbm.at[s], src_v)
        pltpu.sync_copy(dst_hbm.at[s], dst_v)
        if with_counts:
            pltpu.sync_copy(ones_hbm, ones_v)

        def run(x_hbm, do_cnt):
            def body(j, carry):
                gbuf = rows_v.at[pl.ds(0, EC)]
                pltpu.async_copy(x_hbm.at[src_v.at[j]], gbuf, sem).wait()
                pltpu.sync_copy(gbuf, acc_sh.at[dst_v.at[j]], add=True)
                if do_cnt:
                    pltpu.sync_copy(ones_v, cnt_sh.at[dst_v.at[j]], add=True)
                return carry
            lax.fori_loop(0, NCHUNK, body, 0)

        for p in range(2):
            do_cnt = with_counts and p == 0
            pltpu.sync_copy(zq_hbm, rows_v)
            pltpu.sync_copy(rows_v, acc_sh.at[pl.ds(r0, RPT)])
            if do_cnt:
                pltpu.sync_copy(z16_hbm, cbuf_v)
                pltpu.sync_copy(cbuf_v, cnt_sh.at[pl.ds(r0, RPT)])
            plsc.subcore_barrier()

            @pl.when(c == 0)
            def _():
                run(xq[2 * p], do_cnt)

            @pl.when(c == 1)
            def _():
                run(xq[2 * p + 1], False)

            plsc.subcore_barrier()

            pltpu.sync_copy(acc_sh.at[pl.ds(r0, RPT)], rows_v)

            @pl.when(c == 0)
            def _():
                pltpu.sync_copy(rows_v, oq[2 * p].at[pl.ds(r0, RPT)])
                if do_cnt:
                    pltpu.sync_copy(cnt_sh.at[pl.ds(r0, RPT)], cbuf_v)
                    pltpu.sync_copy(cbuf_v, ocnt_hbm.at[pl.ds(r0, RPT)])

            @pl.when(c == 1)
            def _():
                pltpu.sync_copy(rows_v, oq[2 * p + 1].at[pl.ds(r0, RPT)])

    return k(*xqs, src_t, dst_t, zerosq, zeros16, ones16)


def _gate_body(*refs, final):
    (xq0, xq1, xq2, xq3, sq0, sq1, sq2, sq3, cnt_ref, tm_ref,
     twx_ref, twm_ref, tb_ref, g_ref, b_ref) = refs[:15]
    rest = refs[15:]
    if final:
        wout_ref, bout_ref = rest[:2]
        rest = rest[2:]
    o_refs = rest
    x = jnp.concatenate([xq0[...], xq1[...], xq2[...], xq3[...]], axis=-1)
    inv = 1.0 / jnp.maximum(cnt_ref[...][:, 0:1], 1.0)
    m = jnp.concatenate([sq0[...], sq1[...], sq2[...], sq3[...]],
                        axis=-1) * inv
    z = (jnp.dot(x, twx_ref[...], preferred_element_type=jnp.float32)
         + jnp.dot(m, twm_ref[...], preferred_element_type=jnp.float32)
         + tb_ref[...])
    z = z - jnp.max(z, axis=-1, keepdims=True)
    ez = jnp.exp(z)
    raw = ez / jnp.sum(ez, axis=-1, keepdims=True)
    ii = lax.broadcasted_iota(jnp.int32, (CH, CH), 0)
    jj = lax.broadcasted_iota(jnp.int32, (CH, CH), 1)
    tri = (ii <= jj).astype(jnp.float32)
    raw = jnp.dot(raw, tri, preferred_element_type=jnp.float32)
    tm = tm_ref[...]
    raw = tm + (1.0 - tm) * raw
    rr = lax.broadcasted_iota(jnp.int32, (CH, HID), 0)
    cc = lax.broadcasted_iota(jnp.int32, (CH, HID), 1)
    rep = HID // CH
    exp_m = ((cc >= rr * rep) & (cc < rr * rep + rep)).astype(jnp.float32)
    sig = jnp.dot(raw, exp_m, preferred_element_type=jnp.float32)
    xn = x * sig + m * (1.0 - sig)
    xn = _ln(xn, g_ref[...], b_ref[...])
    for i in range(NQ):
        o_refs[i][...] = xn[:, i * Q:(i + 1) * Q]
    o_refs[NQ][...] = raw
    if final:
        o_refs[NQ + 1][...] = (
            jnp.dot(xn, wout_ref[...], preferred_element_type=jnp.float32)
            + bout_ref[...])


def _gate(xqs, sqs, cnt, tm, tw, tb, g, b, wout=None, bout=None):
    final = wout is not None
    grid = N // BLK
    full = lambda r, c: pl.BlockSpec((r, c), lambda i: (0, 0))
    row = lambda c: pl.BlockSpec((BLK, c), lambda i: (i, 0))
    in_specs = ([row(Q)] * NQ + [row(Q)] * NQ
                + [row(16), row(CH), full(HID, CH), full(HID, CH),
                   full(1, CH), full(1, HID), full(1, HID)])
    args = (list(xqs) + list(sqs)
            + [cnt, tm, tw[:HID], tw[HID:], tb.reshape(1, CH),
               g.reshape(1, HID), b.reshape(1, HID)])
    out_specs = [row(Q)] * NQ + [row(CH)]
    out_shape = ([jax.ShapeDtypeStruct((N, Q), jnp.float32)] * NQ
                 + [jax.ShapeDtypeStruct((N, CH), jnp.float32)])
    if final:
        in_specs += [full(HID, OUTD), full(1, OUTD)]
        args += [wout, bout.reshape(1, OUTD)]
        out_specs.append(row(OUTD))
        out_shape.append(jax.ShapeDtypeStruct((N, OUTD), jnp.float32))
    return pl.pallas_call(
        functools.partial(_gate_body, final=final),
        grid=(grid,),
        in_specs=in_specs,
        out_specs=out_specs,
        out_shape=out_shape,
    )(*args)


def _user_gather(y, idx):
    nb = idx.shape[0]
    bpw = nb // (NC * NS)
    mesh = plsc.VectorSubcoreMesh(core_axis_name="c", subcore_axis_name="s")

    @functools.partial(
        pl.kernel, mesh=mesh,
        out_type=jax.ShapeDtypeStruct((nb, OUTD), jnp.float32),
        scratch_types=[
            pltpu.VMEM((bpw,), jnp.int32),
            pltpu.VMEM((bpw, OUTD), jnp.float32),
            pltpu.SemaphoreType.DMA,
        ],
    )
    def k(y_hbm, idx_hbm, out_hbm, idx_v, rows_v, sem):
        wid = lax.axis_index("s") * NC + lax.axis_index("c")
        base = wid * bpw
        pltpu.sync_copy(idx_hbm.at[pl.ds(base, bpw)], idx_v)
        pltpu.async_copy(y_hbm.at[idx_v], rows_v, sem).wait()
        pltpu.sync_copy(rows_v, out_hbm.at[pl.ds(base, bpw)])

    return k(y, idx)


def kernel(user_idx, edge_index, x_table, Win0, bin0, Win1, bin1, ln0_g,
           ln0_b, ln1_g, ln1_b, tm0_W, tm0_b, tm1_W, tm1_b, tmn0_g, tmn0_b,
           tmn1_g, tmn1_b, Wout, bout):
    f32 = jnp.float32
    user_idx = user_idx.astype(jnp.int32)
    src = edge_index[0].astype(jnp.int32)
    dst = edge_index[1].astype(jnp.int32)
    npad = E_PAD - E
    src_t = jnp.concatenate([src, jnp.zeros((npad,), jnp.int32)])
    src_t = src_t.reshape(NS, NCHUNK, EC)
    dst_t = jnp.concatenate([dst, jnp.full((npad,), N, jnp.int32)])
    dst_t = dst_t.reshape(NS, NCHUNK, EC)
    zerosq = jnp.zeros((RPT, Q), f32)
    zeros16 = jnp.zeros((RPT, 16), f32)
    ones16 = jnp.ones((EC, 16), f32)

    xqs = _mlp(x_table, Win0, bin0, Win1, bin1, ln0_g, ln0_b, ln1_g, ln1_b)

    *sqs, cnt = _segsum_pallas(xqs, src_t, dst_t, zerosq, zeros16, ones16,
                               with_counts=True)
    tm0 = jnp.zeros((N, CH), f32)
    *xg1, tm1 = _gate(xqs, sqs, cnt, tm0, tm0_W, tm0_b, tmn0_g, tmn0_b)

    sqs2 = _segsum_pallas(xg1, src_t, dst_t, zerosq, zeros16, ones16,
                          with_counts=False)
    outs = _gate(xg1, sqs2, cnt, tm1, tm1_W, tm1_b, tmn1_g, tmn1_b,
                 wout=Wout, bout=bout)
    y = outs[-1]

    user_embedding = _user_gather(y, user_idx)
    item_embedding = y[USERS:]
    return (user_embedding, item_embedding)


# speedup vs baseline: 3.2066x; 1.1724x over previous
import functools

import jax
import jax.numpy as jnp
from jax import lax
from jax.experimental import pallas as pl
from jax.experimental.pallas import tpu as pltpu
from jax.experimental.pallas import tpu_sc as plsc

N = 10000
USERS = 2000
HID = 256
Q = 64
NQ = 4
CH = 64
OUTD = 128
E = 160000

NC = 2
NS = 16
EC = 128
NCHUNK = 80
E_PAD = NS * NCHUNK * EC
N_PAD = 10112
RPT = N_PAD // NS

BLK = 1000


def _ln(x, g, b, eps=1e-5):
    mu = jnp.mean(x, axis=-1, keepdims=True)
    var = jnp.mean((x - mu) ** 2, axis=-1, keepdims=True)
    return (x - mu) / jnp.sqrt(var + eps) * g + b


def _mlp_body(x_ref, w0_ref, b0_ref, w1_ref, b1_ref, g0_ref, bb0_ref,
              g1_ref, bb1_ref, *o_refs):
    x = x_ref[...]
    h = jnp.maximum(jnp.dot(x, w0_ref[...], preferred_element_type=jnp.float32)
                    + b0_ref[...], 0.0)
    h = _ln(h, g0_ref[...], bb0_ref[...])
    h = jnp.maximum(jnp.dot(h, w1_ref[...], preferred_element_type=jnp.float32)
                    + b1_ref[...], 0.0)
    h = _ln(h, g1_ref[...], bb1_ref[...])
    for i in range(NQ):
        o_refs[i][...] = h[:, i * Q:(i + 1) * Q]


def _mlp(x, w0, b0, w1, b1, g0, bb0, g1, bb1):
    grid = N // BLK
    full = lambda r, c: pl.BlockSpec((r, c), lambda i: (0, 0))
    return pl.pallas_call(
        _mlp_body,
        grid=(grid,),
        in_specs=[
            pl.BlockSpec((BLK, HID), lambda i: (i, 0)),
            full(HID, HID), full(1, HID), full(HID, HID), full(1, HID),
            full(1, HID), full(1, HID), full(1, HID), full(1, HID),
        ],
        out_specs=[pl.BlockSpec((BLK, Q), lambda i: (i, 0))] * NQ,
        out_shape=[jax.ShapeDtypeStruct((N, Q), jnp.float32)] * NQ,
    )(x, w0, b0.reshape(1, HID), w1, b1.reshape(1, HID),
      g0.reshape(1, HID), bb0.reshape(1, HID), g1.reshape(1, HID),
      bb1.reshape(1, HID))


def _segsum_pallas(xqs, src_t, dst_t, zerosq, zeros16, ones16, with_counts):
    out_type = [jax.ShapeDtypeStruct((N_PAD, Q), jnp.float32)] * NQ
    if with_counts:
        out_type = out_type + [jax.ShapeDtypeStruct((N_PAD, 16), jnp.float32)]
    G = 4
    scratch = [
        pltpu.VMEM((NCHUNK, EC), jnp.int32),
        pltpu.VMEM((NCHUNK, EC), jnp.int32),
        pltpu.VMEM((EC, 16), jnp.float32),
        pltpu.VMEM((EC, Q), jnp.float32),
        pltpu.VMEM((EC, 16), jnp.float32),
        pltpu.VMEM_SHARED((N_PAD, Q), jnp.float32),
        pltpu.VMEM_SHARED((N_PAD, 16), jnp.float32),
    ] + [pltpu.VMEM((EC, Q), jnp.float32)] * G \
      + [pltpu.SemaphoreType.DMA] * (G + 1)
    mesh = plsc.VectorSubcoreMesh(core_axis_name="c", subcore_axis_name="s")

    @functools.partial(pl.kernel, out_type=out_type, mesh=mesh,
                       scratch_types=scratch,
                       compiler_params=pltpu.CompilerParams(
                           use_tc_tiling_on_sc=False))
    def k(xq0, xq1, xq2, xq3, src_hbm, dst_hbm, zq_hbm, z16_hbm, ones_hbm,
          oq0, oq1, oq2, oq3, *rest):
        if with_counts:
            ocnt_hbm = rest[0]
            rest = rest[1:]
        src_v, dst_v, ones_v, bnc, cbnc, acc_sh, cnt_sh = rest[:7]
        gb = rest[7:7 + G]
        gsem = rest[7 + G:7 + 2 * G]
        ssem = rest[7 + 2 * G]
        xq = (xq0, xq1, xq2, xq3)
        oq = (oq0, oq1, oq2, oq3)
        c = lax.axis_index("c")
        s = lax.axis_index("s")
        r0 = s * RPT
        NT, TAIL = RPT // EC, RPT % EC

        pltpu.sync_copy(src_hbm.at[s], src_v)
        pltpu.sync_copy(dst_hbm.at[s], dst_v)
        if with_counts:
            pltpu.sync_copy(ones_hbm, ones_v)

        def chunk_fill(dst_sh, zsrc_hbm):
            pltpu.sync_copy(zsrc_hbm, bnc if dst_sh is acc_sh else cbnc)
            zb = bnc if dst_sh is acc_sh else cbnc

            def zf(t, carry):
                pltpu.sync_copy(zb, dst_sh.at[pl.ds(r0 + t * EC, EC)])
                return carry
            lax.fori_loop(0, NT, zf, 0)
            pltpu.sync_copy(zb.at[pl.ds(0, TAIL)],
                            dst_sh.at[pl.ds(r0 + NT * EC, TAIL)])

        def run(x_hbm, do_cnt):
            def body(it, carry):
                j0 = it * G
                gds = [pltpu.async_copy(x_hbm.at[src_v.at[j0 + b]], gb[b],
                                        gsem[b]) for b in range(G)]
                sds = []
                for b in range(G):
                    gds[b].wait()
                    sds.append(pltpu.async_copy(
                        gb[b], acc_sh.at[dst_v.at[j0 + b]], ssem, add=True))
                    if do_cnt:
                        sds.append(pltpu.async_copy(
                            ones_v, cnt_sh.at[dst_v.at[j0 + b]], ssem,
                            add=True))
                for d in sds:
                    d.wait()
                return carry
            lax.fori_loop(0, NCHUNK // G, body, 0)

        def chunk_out(src_sh, zb, out_lo, out_hi):
            def cob(t, carry):
                off = r0 + t * EC
                pltpu.sync_copy(src_sh.at[pl.ds(off, EC)], zb)

                @pl.when(c == 0)
                def _():
                    pltpu.sync_copy(zb, out_lo.at[pl.ds(off, EC)])

                if out_hi is not None:
                    @pl.when(c == 1)
                    def _():
                        pltpu.sync_copy(zb, out_hi.at[pl.ds(off, EC)])
                return carry
            lax.fori_loop(0, NT, cob, 0)
            off = r0 + NT * EC
            pltpu.sync_copy(src_sh.at[pl.ds(off, TAIL)],
                            zb.at[pl.ds(0, TAIL)])

            @pl.when(c == 0)
            def _():
                pltpu.sync_copy(zb.at[pl.ds(0, TAIL)],
                                out_lo.at[pl.ds(off, TAIL)])

            if out_hi is not None:
                @pl.when(c == 1)
                def _():
                    pltpu.sync_copy(zb.at[pl.ds(0, TAIL)],
                                    out_hi.at[pl.ds(off, TAIL)])

        for p in range(2):
            do_cnt = with_counts and p == 0
            chunk_fill(acc_sh, zq_hbm)
            if do_cnt:
                chunk_fill(cnt_sh, z16_hbm)
            plsc.subcore_barrier()

            @pl.when(c == 0)
            def _():
                run(xq[2 * p], do_cnt)

            @pl.when(c == 1)
            def _():
                run(xq[2 * p + 1], False)

            plsc.subcore_barrier()

            chunk_out(acc_sh, bnc, oq[2 * p], oq[2 * p + 1])
            if do_cnt:
                chunk_out(cnt_sh, cbnc, ocnt_hbm, None)

    return k(*xqs, src_t, dst_t, zerosq, zeros16, ones16)


def _gate_body(*refs, final):
    (xq0, xq1, xq2, xq3, sq0, sq1, sq2, sq3, cnt_ref, tm_ref,
     twx_ref, twm_ref, tb_ref, g_ref, b_ref) = refs[:15]
    rest = refs[15:]
    if final:
        wout_ref, bout_ref = rest[:2]
        rest = rest[2:]
    o_refs = rest
    x = jnp.concatenate([xq0[...], xq1[...], xq2[...], xq3[...]], axis=-1)
    inv = 1.0 / jnp.maximum(cnt_ref[...][:, 0:1], 1.0)
    m = jnp.concatenate([sq0[...], sq1[...], sq2[...], sq3[...]],
                        axis=-1) * inv
    z = (jnp.dot(x, twx_ref[...], preferred_element_type=jnp.float32)
         + jnp.dot(m, twm_ref[...], preferred_element_type=jnp.float32)
         + tb_ref[...])
    z = z - jnp.max(z, axis=-1, keepdims=True)
    ez = jnp.exp(z)
    raw = ez / jnp.sum(ez, axis=-1, keepdims=True)
    ii = lax.broadcasted_iota(jnp.int32, (CH, CH), 0)
    jj = lax.broadcasted_iota(jnp.int32, (CH, CH), 1)
    tri = (ii <= jj).astype(jnp.float32)
    raw = jnp.dot(raw, tri, preferred_element_type=jnp.float32)
    tm = tm_ref[...]
    raw = tm + (1.0 - tm) * raw
    rr = lax.broadcasted_iota(jnp.int32, (CH, HID), 0)
    cc = lax.broadcasted_iota(jnp.int32, (CH, HID), 1)
    rep = HID // CH
    exp_m = ((cc >= rr * rep) & (cc < rr * rep + rep)).astype(jnp.float32)
    sig = jnp.dot(raw, exp_m, preferred_element_type=jnp.float32)
    xn = x * sig + m * (1.0 - sig)
    xn = _ln(xn, g_ref[...], b_ref[...])
    for i in range(NQ):
        o_refs[i][...] = xn[:, i * Q:(i + 1) * Q]
    o_refs[NQ][...] = raw
    if final:
        o_refs[NQ + 1][...] = (
            jnp.dot(xn, wout_ref[...], preferred_element_type=jnp.float32)
            + bout_ref[...])


def _gate(xqs, sqs, cnt, tm, tw, tb, g, b, wout=None, bout=None):
    final = wout is not None
    grid = N // BLK
    full = lambda r, c: pl.BlockSpec((r, c), lambda i: (0, 0))
    row = lambda c: pl.BlockSpec((BLK, c), lambda i: (i, 0))
    in_specs = ([row(Q)] * NQ + [row(Q)] * NQ
                + [row(16), row(CH), full(HID, CH), full(HID, CH),
                   full(1, CH), full(1, HID), full(1, HID)])
    args = (list(xqs) + list(sqs)
            + [cnt, tm, tw[:HID], tw[HID:], tb.reshape(1, CH),
               g.reshape(1, HID), b.reshape(1, HID)])
    out_specs = [row(Q)] * NQ + [row(CH)]
    out_shape = ([jax.ShapeDtypeStruct((N, Q), jnp.float32)] * NQ
                 + [jax.ShapeDtypeStruct((N, CH), jnp.float32)])
    if final:
        in_specs += [full(HID, OUTD), full(1, OUTD)]
        args += [wout, bout.reshape(1, OUTD)]
        out_specs.append(row(OUTD))
        out_shape.append(jax.ShapeDtypeStruct((N, OUTD), jnp.float32))
    return pl.pallas_call(
        functools.partial(_gate_body, final=final),
        grid=(grid,),
        in_specs=in_specs,
        out_specs=out_specs,
        out_shape=out_shape,
    )(*args)


def _user_gather(y, idx):
    nb = idx.shape[0]
    bpw = nb // (NC * NS)
    mesh = plsc.VectorSubcoreMesh(core_axis_name="c", subcore_axis_name="s")

    @functools.partial(
        pl.kernel, mesh=mesh,
        out_type=jax.ShapeDtypeStruct((nb, OUTD), jnp.float32),
        scratch_types=[
            pltpu.VMEM((bpw,), jnp.int32),
            pltpu.VMEM((bpw, OUTD), jnp.float32),
            pltpu.SemaphoreType.DMA,
        ],
    )
    def k(y_hbm, idx_hbm, out_hbm, idx_v, rows_v, sem):
        wid = lax.axis_index("s") * NC + lax.axis_index("c")
        base = wid * bpw
        pltpu.sync_copy(idx_hbm.at[pl.ds(base, bpw)], idx_v)
        pltpu.async_copy(y_hbm.at[idx_v], rows_v, sem).wait()
        pltpu.sync_copy(rows_v, out_hbm.at[pl.ds(base, bpw)])

    return k(y, idx)


def kernel(user_idx, edge_index, x_table, Win0, bin0, Win1, bin1, ln0_g,
           ln0_b, ln1_g, ln1_b, tm0_W, tm0_b, tm1_W, tm1_b, tmn0_g, tmn0_b,
           tmn1_g, tmn1_b, Wout, bout):
    f32 = jnp.float32
    user_idx = user_idx.astype(jnp.int32)
    src = edge_index[0].astype(jnp.int32)
    dst = edge_index[1].astype(jnp.int32)
    npad = E_PAD - E
    src_t = jnp.concatenate([src, jnp.zeros((npad,), jnp.int32)])
    src_t = src_t.reshape(NS, NCHUNK, EC)
    dst_t = jnp.concatenate([dst, jnp.full((npad,), N, jnp.int32)])
    dst_t = dst_t.reshape(NS, NCHUNK, EC)
    zerosq = jnp.zeros((EC, Q), f32)
    zeros16 = jnp.zeros((EC, 16), f32)
    ones16 = jnp.ones((EC, 16), f32)

    xqs = _mlp(x_table, Win0, bin0, Win1, bin1, ln0_g, ln0_b, ln1_g, ln1_b)

    *sqs, cnt = _segsum_pallas(xqs, src_t, dst_t, zerosq, zeros16, ones16,
                               with_counts=True)
    tm0 = jnp.zeros((N, CH), f32)
    *xg1, tm1 = _gate(xqs, sqs, cnt, tm0, tm0_W, tm0_b, tmn0_g, tmn0_b)

    sqs2 = _segsum_pallas(xg1, src_t, dst_t, zerosq, zeros16, ones16,
                          with_counts=False)
    outs = _gate(xg1, sqs2, cnt, tm1, tm1_W, tm1_b, tmn1_g, tmn1_b,
                 wout=Wout, bout=bout)
    y = outs[-1]

    user_embedding = _user_gather(y, user_idx)
    item_embedding = y[USERS:]
    return (user_embedding, item_embedding)


# speedup vs baseline: 3.4864x; 1.0873x over previous
import functools

import jax
import jax.numpy as jnp
from jax import lax
from jax.experimental import pallas as pl
from jax.experimental.pallas import tpu as pltpu
from jax.experimental.pallas import tpu_sc as plsc

N = 10000
USERS = 2000
HID = 256
Q = 64
NQ = 4
CH = 64
OUTD = 128
E = 160000

NC = 2
NS = 16
EC = 128
NCHUNK = 80
E_PAD = NS * NCHUNK * EC
N_PAD = 10112
RPT = N_PAD // NS

BLK = 1000


def _ln(x, g, b, eps=1e-5):
    mu = jnp.mean(x, axis=-1, keepdims=True)
    var = jnp.mean((x - mu) ** 2, axis=-1, keepdims=True)
    return (x - mu) / jnp.sqrt(var + eps) * g + b


def _mlp_body(x_ref, w0_ref, b0_ref, w1_ref, b1_ref, g0_ref, bb0_ref,
              g1_ref, bb1_ref, *o_refs):
    x = x_ref[...]
    h = jnp.maximum(jnp.dot(x, w0_ref[...], preferred_element_type=jnp.float32)
                    + b0_ref[...], 0.0)
    h = _ln(h, g0_ref[...], bb0_ref[...])
    h = jnp.maximum(jnp.dot(h, w1_ref[...], preferred_element_type=jnp.float32)
                    + b1_ref[...], 0.0)
    h = _ln(h, g1_ref[...], bb1_ref[...])
    for i in range(NQ):
        o_refs[i][...] = h[:, i * Q:(i + 1) * Q]


def _mlp(x, w0, b0, w1, b1, g0, bb0, g1, bb1):
    grid = N // BLK
    full = lambda r, c: pl.BlockSpec((r, c), lambda i: (0, 0))
    return pl.pallas_call(
        _mlp_body,
        grid=(grid,),
        in_specs=[
            pl.BlockSpec((BLK, HID), lambda i: (i, 0)),
            full(HID, HID), full(1, HID), full(HID, HID), full(1, HID),
            full(1, HID), full(1, HID), full(1, HID), full(1, HID),
        ],
        out_specs=[pl.BlockSpec((BLK, Q), lambda i: (i, 0))] * NQ,
        out_shape=[jax.ShapeDtypeStruct((N, Q), jnp.float32)] * NQ,
    )(x, w0, b0.reshape(1, HID), w1, b1.reshape(1, HID),
      g0.reshape(1, HID), bb0.reshape(1, HID), g1.reshape(1, HID),
      bb1.reshape(1, HID))


def _segsum_pallas(xqs, src_t, dst_t, zerosq, zeros16, ones16, with_counts):
    out_type = [jax.ShapeDtypeStruct((N_PAD, Q), jnp.float32)] * NQ
    if with_counts:
        out_type = out_type + [jax.ShapeDtypeStruct((N_PAD, 16), jnp.float32)]
    G = 5
    scratch = [
        pltpu.VMEM((NCHUNK, EC), jnp.int32),
        pltpu.VMEM((NCHUNK, EC), jnp.int32),
        pltpu.VMEM((EC, 16), jnp.float32),
        pltpu.VMEM((EC, Q), jnp.float32),
        pltpu.VMEM((EC, 16), jnp.float32),
        pltpu.VMEM_SHARED((N_PAD, Q), jnp.float32),
        pltpu.VMEM_SHARED((N_PAD, 16), jnp.float32),
    ] + [pltpu.VMEM((EC, Q), jnp.float32)] * G \
      + [pltpu.SemaphoreType.DMA] * (2 * G)
    mesh = plsc.VectorSubcoreMesh(core_axis_name="c", subcore_axis_name="s")

    @functools.partial(pl.kernel, out_type=out_type, mesh=mesh,
                       scratch_types=scratch,
                       compiler_params=pltpu.CompilerParams(
                           use_tc_tiling_on_sc=False))
    def k(xq0, xq1, xq2, xq3, src_hbm, dst_hbm, zq_hbm, z16_hbm, ones_hbm,
          oq0, oq1, oq2, oq3, *rest):
        if with_counts:
            ocnt_hbm = rest[0]
            rest = rest[1:]
        src_v, dst_v, ones_v, bnc, cbnc, acc_sh, cnt_sh = rest[:7]
        gb = rest[7:7 + G]
        gsem = rest[7 + G:7 + 2 * G]
        ssems = rest[7 + 2 * G:7 + 3 * G]
        xq = (xq0, xq1, xq2, xq3)
        oq = (oq0, oq1, oq2, oq3)
        c = lax.axis_index("c")
        s = lax.axis_index("s")
        r0 = s * RPT
        NT, TAIL = RPT // EC, RPT % EC

        pltpu.sync_copy(src_hbm.at[s], src_v)
        pltpu.sync_copy(dst_hbm.at[s], dst_v)
        if with_counts:
            pltpu.sync_copy(ones_hbm, ones_v)

        def chunk_fill(dst_sh, zsrc_hbm):
            pltpu.sync_copy(zsrc_hbm, bnc if dst_sh is acc_sh else cbnc)
            zb = bnc if dst_sh is acc_sh else cbnc

            def zf(t, carry):
                pltpu.sync_copy(zb, dst_sh.at[pl.ds(r0 + t * EC, EC)])
                return carry
            lax.fori_loop(0, NT, zf, 0)
            pltpu.sync_copy(zb.at[pl.ds(0, TAIL)],
                            dst_sh.at[pl.ds(r0 + NT * EC, TAIL)])

        def run(x_hbm, do_cnt):
            def wait_scatter(b):
                pltpu.make_async_copy(
                    gb[b], acc_sh.at[dst_v.at[0]], ssems[b]).wait()
                if do_cnt:
                    pltpu.make_async_copy(
                        ones_v, cnt_sh.at[dst_v.at[0]], ssems[b]).wait()

            def body(it, carry):
                j0 = it * G
                gds = []
                for b in range(G):
                    @pl.when(it > 0)
                    def _(b=b):
                        wait_scatter(b)
                    gds.append(pltpu.async_copy(
                        x_hbm.at[src_v.at[j0 + b]], gb[b], gsem[b]))
                for b in range(G):
                    gds[b].wait()
                    pltpu.async_copy(
                        gb[b], acc_sh.at[dst_v.at[j0 + b]], ssems[b],
                        add=True)
                    if do_cnt:
                        pltpu.async_copy(
                            ones_v, cnt_sh.at[dst_v.at[j0 + b]], ssems[b],
                            add=True)
                return carry
            lax.fori_loop(0, NCHUNK // G, body, 0)
            for b in range(G):
                wait_scatter(b)

        def chunk_out(src_sh, zb, out_lo, out_hi):
            def cob(t, carry):
                off = r0 + t * EC
                pltpu.sync_copy(src_sh.at[pl.ds(off, EC)], zb)

                @pl.when(c == 0)
                def _():
                    pltpu.sync_copy(zb, out_lo.at[pl.ds(off, EC)])

                if out_hi is not None:
                    @pl.when(c == 1)
                    def _():
                        pltpu.sync_copy(zb, out_hi.at[pl.ds(off, EC)])
                return carry
            lax.fori_loop(0, NT, cob, 0)
            off = r0 + NT * EC
            pltpu.sync_copy(src_sh.at[pl.ds(off, TAIL)],
                            zb.at[pl.ds(0, TAIL)])

            @pl.when(c == 0)
            def _():
                pltpu.sync_copy(zb.at[pl.ds(0, TAIL)],
                                out_lo.at[pl.ds(off, TAIL)])

            if out_hi is not None:
                @pl.when(c == 1)
                def _():
                    pltpu.sync_copy(zb.at[pl.ds(0, TAIL)],
                                    out_hi.at[pl.ds(off, TAIL)])

        for p in range(2):
            do_cnt = with_counts and p == 0
            chunk_fill(acc_sh, zq_hbm)
            if do_cnt:
                chunk_fill(cnt_sh, z16_hbm)
            plsc.subcore_barrier()

            @pl.when(c == 0)
            def _():
                run(xq[2 * p], do_cnt)

            @pl.when(c == 1)
            def _():
                run(xq[2 * p + 1], False)

            plsc.subcore_barrier()

            chunk_out(acc_sh, bnc, oq[2 * p], oq[2 * p + 1])
            if do_cnt:
                chunk_out(cnt_sh, cbnc, ocnt_hbm, None)

    return k(*xqs, src_t, dst_t, zerosq, zeros16, ones16)


def _gate_body(*refs, final):
    (xq0, xq1, xq2, xq3, sq0, sq1, sq2, sq3, cnt_ref, tm_ref,
     twx_ref, twm_ref, tb_ref, g_ref, b_ref) = refs[:15]
    rest = refs[15:]
    if final:
        wout_ref, bout_ref = rest[:2]
        rest = rest[2:]
    o_refs = rest
    x = jnp.concatenate([xq0[...], xq1[...], xq2[...], xq3[...]], axis=-1)
    inv = 1.0 / jnp.maximum(cnt_ref[...][:, 0:1], 1.0)
    m = jnp.concatenate([sq0[...], sq1[...], sq2[...], sq3[...]],
                        axis=-1) * inv
    z = (jnp.dot(x, twx_ref[...], preferred_element_type=jnp.float32)
         + jnp.dot(m, twm_ref[...], preferred_element_type=jnp.float32)
         + tb_ref[...])
    z = z - jnp.max(z, axis=-1, keepdims=True)
    ez = jnp.exp(z)
    raw = ez / jnp.sum(ez, axis=-1, keepdims=True)
    ii = lax.broadcasted_iota(jnp.int32, (CH, CH), 0)
    jj = lax.broadcasted_iota(jnp.int32, (CH, CH), 1)
    tri = (ii <= jj).astype(jnp.float32)
    raw = jnp.dot(raw, tri, preferred_element_type=jnp.float32)
    tm = tm_ref[...]
    raw = tm + (1.0 - tm) * raw
    rr = lax.broadcasted_iota(jnp.int32, (CH, HID), 0)
    cc = lax.broadcasted_iota(jnp.int32, (CH, HID), 1)
    rep = HID // CH
    exp_m = ((cc >= rr * rep) & (cc < rr * rep + rep)).astype(jnp.float32)
    sig = jnp.dot(raw, exp_m, preferred_element_type=jnp.float32)
    xn = x * sig + m * (1.0 - sig)
    xn = _ln(xn, g_ref[...], b_ref[...])
    for i in range(NQ):
        o_refs[i][...] = xn[:, i * Q:(i + 1) * Q]
    o_refs[NQ][...] = raw
    if final:
        o_refs[NQ + 1][...] = (
            jnp.dot(xn, wout_ref[...], preferred_element_type=jnp.float32)
            + bout_ref[...])


def _gate(xqs, sqs, cnt, tm, tw, tb, g, b, wout=None, bout=None):
    final = wout is not None
    grid = N // BLK
    full = lambda r, c: pl.BlockSpec((r, c), lambda i: (0, 0))
    row = lambda c: pl.BlockSpec((BLK, c), lambda i: (i, 0))
    in_specs = ([row(Q)] * NQ + [row(Q)] * NQ
                + [row(16), row(CH), full(HID, CH), full(HID, CH),
                   full(1, CH), full(1, HID), full(1, HID)])
    args = (list(xqs) + list(sqs)
            + [cnt, tm, tw[:HID], tw[HID:], tb.reshape(1, CH),
               g.reshape(1, HID), b.reshape(1, HID)])
    out_specs = [row(Q)] * NQ + [row(CH)]
    out_shape = ([jax.ShapeDtypeStruct((N, Q), jnp.float32)] * NQ
                 + [jax.ShapeDtypeStruct((N, CH), jnp.float32)])
    if final:
        in_specs += [full(HID, OUTD), full(1, OUTD)]
        args += [wout, bout.reshape(1, OUTD)]
        out_specs.append(row(OUTD))
        out_shape.append(jax.ShapeDtypeStruct((N, OUTD), jnp.float32))
    return pl.pallas_call(
        functools.partial(_gate_body, final=final),
        grid=(grid,),
        in_specs=in_specs,
        out_specs=out_specs,
        out_shape=out_shape,
    )(*args)


def _user_gather(y, idx):
    nb = idx.shape[0]
    bpw = nb // (NC * NS)
    mesh = plsc.VectorSubcoreMesh(core_axis_name="c", subcore_axis_name="s")

    @functools.partial(
        pl.kernel, mesh=mesh,
        out_type=jax.ShapeDtypeStruct((nb, OUTD), jnp.float32),
        scratch_types=[
            pltpu.VMEM((bpw,), jnp.int32),
            pltpu.VMEM((bpw, OUTD), jnp.float32),
            pltpu.SemaphoreType.DMA,
        ],
    )
    def k(y_hbm, idx_hbm, out_hbm, idx_v, rows_v, sem):
        wid = lax.axis_index("s") * NC + lax.axis_index("c")
        base = wid * bpw
        pltpu.sync_copy(idx_hbm.at[pl.ds(base, bpw)], idx_v)
        pltpu.async_copy(y_hbm.at[idx_v], rows_v, sem).wait()
        pltpu.sync_copy(rows_v, out_hbm.at[pl.ds(base, bpw)])

    return k(y, idx)


def kernel(user_idx, edge_index, x_table, Win0, bin0, Win1, bin1, ln0_g,
           ln0_b, ln1_g, ln1_b, tm0_W, tm0_b, tm1_W, tm1_b, tmn0_g, tmn0_b,
           tmn1_g, tmn1_b, Wout, bout):
    f32 = jnp.float32
    user_idx = user_idx.astype(jnp.int32)
    src = edge_index[0].astype(jnp.int32)
    dst = edge_index[1].astype(jnp.int32)
    npad = E_PAD - E
    src_t = jnp.concatenate([src, jnp.zeros((npad,), jnp.int32)])
    src_t = src_t.reshape(NS, NCHUNK, EC)
    dst_t = jnp.concatenate([dst, jnp.full((npad,), N, jnp.int32)])
    dst_t = dst_t.reshape(NS, NCHUNK, EC)
    zerosq = jnp.zeros((EC, Q), f32)
    zeros16 = jnp.zeros((EC, 16), f32)
    ones16 = jnp.ones((EC, 16), f32)

    xqs = _mlp(x_table, Win0, bin0, Win1, bin1, ln0_g, ln0_b, ln1_g, ln1_b)

    *sqs, cnt = _segsum_pallas(xqs, src_t, dst_t, zerosq, zeros16, ones16,
                               with_counts=True)
    tm0 = jnp.zeros((N, CH), f32)
    *xg1, tm1 = _gate(xqs, sqs, cnt, tm0, tm0_W, tm0_b, tmn0_g, tmn0_b)

    sqs2 = _segsum_pallas(xg1, src_t, dst_t, zerosq, zeros16, ones16,
                          with_counts=False)
    outs = _gate(xg1, sqs2, cnt, tm1, tm1_W, tm1_b, tmn1_g, tmn1_b,
                 wout=Wout, bout=bout)
    y = outs[-1]

    user_embedding = _user_gather(y, user_idx)
    item_embedding = y[USERS:]
    return (user_embedding, item_embedding)


# speedup vs baseline: 3.4864x; 1.0000x over previous
import functools

import jax
import jax.numpy as jnp
from jax import lax
from jax.experimental import pallas as pl
from jax.experimental.pallas import tpu as pltpu
from jax.experimental.pallas import tpu_sc as plsc

N = 10000
USERS = 2000
HID = 256
Q = 64
NQ = 4
CH = 64
OUTD = 128
E = 160000

NC = 2
NS = 16
EC = 128
NCHUNK = 80
E_PAD = NS * NCHUNK * EC
N_PAD = 10112
RPT = N_PAD // NS

BLK = 1000


def _ln(x, g, b, eps=1e-5):
    mu = jnp.mean(x, axis=-1, keepdims=True)
    var = jnp.mean((x - mu) ** 2, axis=-1, keepdims=True)
    return (x - mu) / jnp.sqrt(var + eps) * g + b


def _mlp_body(x_ref, w0_ref, b0_ref, w1_ref, b1_ref, g0_ref, bb0_ref,
              g1_ref, bb1_ref, *o_refs):
    bf = jnp.bfloat16
    x = x_ref[...]
    h = jnp.maximum(
        jnp.dot(x.astype(bf), w0_ref[...].astype(bf),
                preferred_element_type=jnp.float32) + b0_ref[...], 0.0)
    h = _ln(h, g0_ref[...], bb0_ref[...])
    h = jnp.maximum(
        jnp.dot(h.astype(bf), w1_ref[...].astype(bf),
                preferred_element_type=jnp.float32) + b1_ref[...], 0.0)
    h = _ln(h, g1_ref[...], bb1_ref[...])
    for i in range(NQ):
        o_refs[i][...] = h[:, i * Q:(i + 1) * Q]


def _mlp(x, w0, b0, w1, b1, g0, bb0, g1, bb1):
    grid = N // BLK
    full = lambda r, c: pl.BlockSpec((r, c), lambda i: (0, 0))
    return pl.pallas_call(
        _mlp_body,
        grid=(grid,),
        in_specs=[
            pl.BlockSpec((BLK, HID), lambda i: (i, 0)),
            full(HID, HID), full(1, HID), full(HID, HID), full(1, HID),
            full(1, HID), full(1, HID), full(1, HID), full(1, HID),
        ],
        out_specs=[pl.BlockSpec((BLK, Q), lambda i: (i, 0))] * NQ,
        out_shape=[jax.ShapeDtypeStruct((N, Q), jnp.float32)] * NQ,
    )(x, w0, b0.reshape(1, HID), w1, b1.reshape(1, HID),
      g0.reshape(1, HID), bb0.reshape(1, HID), g1.reshape(1, HID),
      bb1.reshape(1, HID))


def _segsum_pallas(xqs, src_t, dst_t, zerosq, zeros16, ones16, with_counts):
    out_type = [jax.ShapeDtypeStruct((N_PAD, Q), jnp.float32)] * NQ
    if with_counts:
        out_type = out_type + [jax.ShapeDtypeStruct((N_PAD, 16), jnp.float32)]
    G = 5
    scratch = [
        pltpu.VMEM((NCHUNK, EC), jnp.int32),
        pltpu.VMEM((NCHUNK, EC), jnp.int32),
        pltpu.VMEM((EC, 16), jnp.float32),
        pltpu.VMEM((EC, Q), jnp.float32),
        pltpu.VMEM((EC, 16), jnp.float32),
        pltpu.VMEM_SHARED((N_PAD, Q), jnp.float32),
        pltpu.VMEM_SHARED((N_PAD, 16), jnp.float32),
    ] + [pltpu.VMEM((EC, Q), jnp.float32)] * G \
      + [pltpu.SemaphoreType.DMA] * (2 * G)
    mesh = plsc.VectorSubcoreMesh(core_axis_name="c", subcore_axis_name="s")

    @functools.partial(pl.kernel, out_type=out_type, mesh=mesh,
                       scratch_types=scratch,
                       compiler_params=pltpu.CompilerParams(
                           use_tc_tiling_on_sc=False))
    def k(xq0, xq1, xq2, xq3, src_hbm, dst_hbm, zq_hbm, z16_hbm, ones_hbm,
          oq0, oq1, oq2, oq3, *rest):
        if with_counts:
            ocnt_hbm = rest[0]
            rest = rest[1:]
        src_v, dst_v, ones_v, bnc, cbnc, acc_sh, cnt_sh = rest[:7]
        gb = rest[7:7 + G]
        gsem = rest[7 + G:7 + 2 * G]
        ssems = rest[7 + 2 * G:7 + 3 * G]
        xq = (xq0, xq1, xq2, xq3)
        oq = (oq0, oq1, oq2, oq3)
        c = lax.axis_index("c")
        s = lax.axis_index("s")
        r0 = s * RPT
        NT, TAIL = RPT // EC, RPT % EC

        pltpu.sync_copy(src_hbm.at[s], src_v)
        pltpu.sync_copy(dst_hbm.at[s], dst_v)
        if with_counts:
            pltpu.sync_copy(ones_hbm, ones_v)

        def chunk_fill(dst_sh, zsrc_hbm):
            pltpu.sync_copy(zsrc_hbm, bnc if dst_sh is acc_sh else cbnc)
            zb = bnc if dst_sh is acc_sh else cbnc

            def zf(t, carry):
                pltpu.sync_copy(zb, dst_sh.at[pl.ds(r0 + t * EC, EC)])
                return carry
            lax.fori_loop(0, NT, zf, 0)
            pltpu.sync_copy(zb.at[pl.ds(0, TAIL)],
                            dst_sh.at[pl.ds(r0 + NT * EC, TAIL)])

        def run(x_hbm, do_cnt):
            def wait_scatter(b):
                pltpu.make_async_copy(
                    gb[b], acc_sh.at[dst_v.at[0]], ssems[b]).wait()
                if do_cnt:
                    pltpu.make_async_copy(
                        ones_v, cnt_sh.at[dst_v.at[0]], ssems[b]).wait()

            def body(it, carry):
                j0 = it * G
                gds = []
                for b in range(G):
                    @pl.when(it > 0)
                    def _(b=b):
                        wait_scatter(b)
                    gds.append(pltpu.async_copy(
                        x_hbm.at[src_v.at[j0 + b]], gb[b], gsem[b]))
                for b in range(G):
                    gds[b].wait()
                    pltpu.async_copy(
                        gb[b], acc_sh.at[dst_v.at[j0 + b]], ssems[b],
                        add=True)
                    if do_cnt:
                        pltpu.async_copy(
                            ones_v, cnt_sh.at[dst_v.at[j0 + b]], ssems[b],
                            add=True)
                return carry
            lax.fori_loop(0, NCHUNK // G, body, 0)
            for b in range(G):
                wait_scatter(b)

        def chunk_out(src_sh, zb, out_lo, out_hi):
            def cob(t, carry):
                off = r0 + t * EC
                pltpu.sync_copy(src_sh.at[pl.ds(off, EC)], zb)

                @pl.when(c == 0)
                def _():
                    pltpu.sync_copy(zb, out_lo.at[pl.ds(off, EC)])

                if out_hi is not None:
                    @pl.when(c == 1)
                    def _():
                        pltpu.sync_copy(zb, out_hi.at[pl.ds(off, EC)])
                return carry
            lax.fori_loop(0, NT, cob, 0)
            off = r0 + NT * EC
            pltpu.sync_copy(src_sh.at[pl.ds(off, TAIL)],
                            zb.at[pl.ds(0, TAIL)])

            @pl.when(c == 0)
            def _():
                pltpu.sync_copy(zb.at[pl.ds(0, TAIL)],
                                out_lo.at[pl.ds(off, TAIL)])

            if out_hi is not None:
                @pl.when(c == 1)
                def _():
                    pltpu.sync_copy(zb.at[pl.ds(0, TAIL)],
                                    out_hi.at[pl.ds(off, TAIL)])

        for p in range(2):
            do_cnt = with_counts and p == 0
            chunk_fill(acc_sh, zq_hbm)
            if do_cnt:
                chunk_fill(cnt_sh, z16_hbm)
            plsc.subcore_barrier()

            @pl.when(c == 0)
            def _():
                run(xq[2 * p], do_cnt)

            @pl.when(c == 1)
            def _():
                run(xq[2 * p + 1], False)

            plsc.subcore_barrier()

            chunk_out(acc_sh, bnc, oq[2 * p], oq[2 * p + 1])
            if do_cnt:
                chunk_out(cnt_sh, cbnc, ocnt_hbm, None)

    return k(*xqs, src_t, dst_t, zerosq, zeros16, ones16)


def _gate_body(*refs, final):
    (xq0, xq1, xq2, xq3, sq0, sq1, sq2, sq3, cnt_ref, tm_ref,
     twx_ref, twm_ref, tb_ref, g_ref, b_ref) = refs[:15]
    rest = refs[15:]
    if final:
        wout_ref, bout_ref = rest[:2]
        rest = rest[2:]
    o_refs = rest
    x = jnp.concatenate([xq0[...], xq1[...], xq2[...], xq3[...]], axis=-1)
    inv = 1.0 / jnp.maximum(cnt_ref[...][:, 0:1], 1.0)
    m = jnp.concatenate([sq0[...], sq1[...], sq2[...], sq3[...]],
                        axis=-1) * inv
    bf = jnp.bfloat16
    z = (jnp.dot(x.astype(bf), twx_ref[...].astype(bf),
                 preferred_element_type=jnp.float32)
         + jnp.dot(m.astype(bf), twm_ref[...].astype(bf),
                   preferred_element_type=jnp.float32)
         + tb_ref[...])
    z = z - jnp.max(z, axis=-1, keepdims=True)
    ez = jnp.exp(z)
    raw = ez / jnp.sum(ez, axis=-1, keepdims=True)
    ii = lax.broadcasted_iota(jnp.int32, (CH, CH), 0)
    jj = lax.broadcasted_iota(jnp.int32, (CH, CH), 1)
    tri = (ii <= jj).astype(jnp.float32)
    raw = jnp.dot(raw, tri, preferred_element_type=jnp.float32)
    tm = tm_ref[...]
    raw = tm + (1.0 - tm) * raw
    rr = lax.broadcasted_iota(jnp.int32, (CH, HID), 0)
    cc = lax.broadcasted_iota(jnp.int32, (CH, HID), 1)
    rep = HID // CH
    exp_m = ((cc >= rr * rep) & (cc < rr * rep + rep)).astype(jnp.float32)
    sig = jnp.dot(raw, exp_m, preferred_element_type=jnp.float32)
    xn = x * sig + m * (1.0 - sig)
    xn = _ln(xn, g_ref[...], b_ref[...])
    for i in range(NQ):
        o_refs[i][...] = xn[:, i * Q:(i + 1) * Q]
    o_refs[NQ][...] = raw
    if final:
        o_refs[NQ + 1][...] = (
            jnp.dot(xn.astype(bf), wout_ref[...].astype(bf),
                    preferred_element_type=jnp.float32) + bout_ref[...])


def _gate(xqs, sqs, cnt, tm, tw, tb, g, b, wout=None, bout=None):
    final = wout is not None
    grid = N // BLK
    full = lambda r, c: pl.BlockSpec((r, c), lambda i: (0, 0))
    row = lambda c: pl.BlockSpec((BLK, c), lambda i: (i, 0))
    in_specs = ([row(Q)] * NQ + [row(Q)] * NQ
                + [row(16), row(CH), full(HID, CH), full(HID, CH),
                   full(1, CH), full(1, HID), full(1, HID)])
    args = (list(xqs) + list(sqs)
            + [cnt, tm, tw[:HID], tw[HID:], tb.reshape(1, CH),
               g.reshape(1, HID), b.reshape(1, HID)])
    out_specs = [row(Q)] * NQ + [row(CH)]
    out_shape = ([jax.ShapeDtypeStruct((N, Q), jnp.float32)] * NQ
                 + [jax.ShapeDtypeStruct((N, CH), jnp.float32)])
    if final:
        in_specs += [full(HID, OUTD), full(1, OUTD)]
        args += [wout, bout.reshape(1, OUTD)]
        out_specs.append(row(OUTD))
        out_shape.append(jax.ShapeDtypeStruct((N, OUTD), jnp.float32))
    return pl.pallas_call(
        functools.partial(_gate_body, final=final),
        grid=(grid,),
        in_specs=in_specs,
        out_specs=out_specs,
        out_shape=out_shape,
    )(*args)


def _user_gather(y, idx):
    nb = idx.shape[0]
    bpw = nb // (NC * NS)
    mesh = plsc.VectorSubcoreMesh(core_axis_name="c", subcore_axis_name="s")

    @functools.partial(
        pl.kernel, mesh=mesh,
        out_type=jax.ShapeDtypeStruct((nb, OUTD), jnp.float32),
        scratch_types=[
            pltpu.VMEM((bpw,), jnp.int32),
            pltpu.VMEM((bpw, OUTD), jnp.float32),
            pltpu.SemaphoreType.DMA,
        ],
    )
    def k(y_hbm, idx_hbm, out_hbm, idx_v, rows_v, sem):
        wid = lax.axis_index("s") * NC + lax.axis_index("c")
        base = wid * bpw
        pltpu.sync_copy(idx_hbm.at[pl.ds(base, bpw)], idx_v)
        pltpu.async_copy(y_hbm.at[idx_v], rows_v, sem).wait()
        pltpu.sync_copy(rows_v, out_hbm.at[pl.ds(base, bpw)])

    return k(y, idx)


def kernel(user_idx, edge_index, x_table, Win0, bin0, Win1, bin1, ln0_g,
           ln0_b, ln1_g, ln1_b, tm0_W, tm0_b, tm1_W, tm1_b, tmn0_g, tmn0_b,
           tmn1_g, tmn1_b, Wout, bout):
    f32 = jnp.float32
    user_idx = user_idx.astype(jnp.int32)
    src = edge_index[0].astype(jnp.int32)
    dst = edge_index[1].astype(jnp.int32)
    npad = E_PAD - E
    src_t = jnp.concatenate([src, jnp.zeros((npad,), jnp.int32)])
    src_t = src_t.reshape(NS, NCHUNK, EC)
    dst_t = jnp.concatenate([dst, jnp.full((npad,), N, jnp.int32)])
    dst_t = dst_t.reshape(NS, NCHUNK, EC)
    zerosq = jnp.zeros((EC, Q), f32)
    zeros16 = jnp.zeros((EC, 16), f32)
    ones16 = jnp.ones((EC, 16), f32)

    xqs = _mlp(x_table, Win0, bin0, Win1, bin1, ln0_g, ln0_b, ln1_g, ln1_b)

    *sqs, cnt = _segsum_pallas(xqs, src_t, dst_t, zerosq, zeros16, ones16,
                               with_counts=True)
    tm0 = jnp.zeros((N, CH), f32)
    *xg1, tm1 = _gate(xqs, sqs, cnt, tm0, tm0_W, tm0_b, tmn0_g, tmn0_b)

    sqs2 = _segsum_pallas(xg1, src_t, dst_t, zerosq, zeros16, ones16,
                          with_counts=False)
    outs = _gate(xg1, sqs2, cnt, tm1, tm1_W, tm1_b, tmn1_g, tmn1_b,
                 wout=Wout, bout=bout)
    y = outs[-1]

    user_embedding = _user_gather(y, user_idx)
    item_embedding = y[USERS:]
    return (user_embedding, item_embedding)


# speedup vs baseline: 3.5665x; 1.0230x over previous
import functools

import jax
import jax.numpy as jnp
from jax import lax
from jax.experimental import pallas as pl
from jax.experimental.pallas import tpu as pltpu
from jax.experimental.pallas import tpu_sc as plsc

N = 10000
USERS = 2000
HID = 256
Q = 64
NQ = 4
CH = 64
OUTD = 128
E = 160000

NC = 2
NS = 16
EC = 128
NCHUNK = 80
E_PAD = NS * NCHUNK * EC
N_PAD = 10112
RPT = N_PAD // NS

BLK = 1000


def _ln(x, g, b, eps=1e-5):
    mu = jnp.mean(x, axis=-1, keepdims=True)
    var = jnp.mean((x - mu) ** 2, axis=-1, keepdims=True)
    return (x - mu) / jnp.sqrt(var + eps) * g + b


def _mlp_body(x_ref, w0_ref, b0_ref, w1_ref, b1_ref, g0_ref, bb0_ref,
              g1_ref, bb1_ref, *o_refs):
    bf = jnp.bfloat16
    x = x_ref[...]
    h = jnp.maximum(
        jnp.dot(x.astype(bf), w0_ref[...].astype(bf),
                preferred_element_type=jnp.float32) + b0_ref[...], 0.0)
    h = _ln(h, g0_ref[...], bb0_ref[...])
    h = jnp.maximum(
        jnp.dot(h.astype(bf), w1_ref[...].astype(bf),
                preferred_element_type=jnp.float32) + b1_ref[...], 0.0)
    h = _ln(h, g1_ref[...], bb1_ref[...])
    for i in range(NQ):
        o_refs[i][...] = h[:, i * Q:(i + 1) * Q]


def _mlp(x, w0, b0, w1, b1, g0, bb0, g1, bb1):
    grid = N // BLK
    full = lambda r, c: pl.BlockSpec((r, c), lambda i: (0, 0))
    return pl.pallas_call(
        _mlp_body,
        grid=(grid,),
        in_specs=[
            pl.BlockSpec((BLK, HID), lambda i: (i, 0)),
            full(HID, HID), full(1, HID), full(HID, HID), full(1, HID),
            full(1, HID), full(1, HID), full(1, HID), full(1, HID),
        ],
        out_specs=[pl.BlockSpec((BLK, Q), lambda i: (i, 0))] * NQ,
        out_shape=[jax.ShapeDtypeStruct((N, Q), jnp.float32)] * NQ,
    )(x, w0, b0.reshape(1, HID), w1, b1.reshape(1, HID),
      g0.reshape(1, HID), bb0.reshape(1, HID), g1.reshape(1, HID),
      bb1.reshape(1, HID))


def _segsum_pallas(xqs, src_t, dst_t, zerosq):
    out_type = [jax.ShapeDtypeStruct((N_PAD, Q), jnp.float32)] * NQ
    G = 5
    scratch = [
        pltpu.VMEM((NCHUNK, EC), jnp.int32),
        pltpu.VMEM((NCHUNK, EC), jnp.int32),
        pltpu.VMEM((EC, Q), jnp.float32),
        pltpu.VMEM_SHARED((N_PAD, Q), jnp.float32),
    ] + [pltpu.VMEM((EC, Q), jnp.float32)] * G \
      + [pltpu.SemaphoreType.DMA] * (2 * G)
    mesh = plsc.VectorSubcoreMesh(core_axis_name="c", subcore_axis_name="s")

    @functools.partial(pl.kernel, out_type=out_type, mesh=mesh,
                       scratch_types=scratch,
                       compiler_params=pltpu.CompilerParams(
                           use_tc_tiling_on_sc=False))
    def k(xq0, xq1, xq2, xq3, src_hbm, dst_hbm, zq_hbm,
          oq0, oq1, oq2, oq3, *rest):
        src_v, dst_v, bnc, acc_sh = rest[:4]
        gb = rest[4:4 + G]
        gsem = rest[4 + G:4 + 2 * G]
        ssems = rest[4 + 2 * G:4 + 3 * G]
        xq = (xq0, xq1, xq2, xq3)
        oq = (oq0, oq1, oq2, oq3)
        c = lax.axis_index("c")
        s = lax.axis_index("s")
        r0 = s * RPT
        NT, TAIL = RPT // EC, RPT % EC

        pltpu.sync_copy(src_hbm.at[s], src_v)
        pltpu.sync_copy(dst_hbm.at[s], dst_v)

        def chunk_fill():
            pltpu.sync_copy(zq_hbm, bnc)

            def zf(t, carry):
                pltpu.sync_copy(bnc, acc_sh.at[pl.ds(r0 + t * EC, EC)])
                return carry
            lax.fori_loop(0, NT, zf, 0)
            pltpu.sync_copy(bnc.at[pl.ds(0, TAIL)],
                            acc_sh.at[pl.ds(r0 + NT * EC, TAIL)])

        def run(x_hbm):
            def wait_scatter(b):
                pltpu.make_async_copy(
                    gb[b], acc_sh.at[dst_v.at[0]], ssems[b]).wait()

            def body(it, carry):
                j0 = it * G
                gds = []
                for b in range(G):
                    @pl.when(it > 0)
                    def _(b=b):
                        wait_scatter(b)
                    gds.append(pltpu.async_copy(
                        x_hbm.at[src_v.at[j0 + b]], gb[b], gsem[b]))
                for b in range(G):
                    gds[b].wait()
                    pltpu.async_copy(
                        gb[b], acc_sh.at[dst_v.at[j0 + b]], ssems[b],
                        add=True)
                return carry
            lax.fori_loop(0, NCHUNK // G, body, 0)
            for b in range(G):
                wait_scatter(b)

        def chunk_out(out_lo, out_hi):
            def cob(t, carry):
                off = r0 + t * EC
                pltpu.sync_copy(acc_sh.at[pl.ds(off, EC)], bnc)

                @pl.when(c == 0)
                def _():
                    pltpu.sync_copy(bnc, out_lo.at[pl.ds(off, EC)])

                @pl.when(c == 1)
                def _():
                    pltpu.sync_copy(bnc, out_hi.at[pl.ds(off, EC)])
                return carry
            lax.fori_loop(0, NT, cob, 0)
            off = r0 + NT * EC
            pltpu.sync_copy(acc_sh.at[pl.ds(off, TAIL)],
                            bnc.at[pl.ds(0, TAIL)])

            @pl.when(c == 0)
            def _():
                pltpu.sync_copy(bnc.at[pl.ds(0, TAIL)],
                                out_lo.at[pl.ds(off, TAIL)])

            @pl.when(c == 1)
            def _():
                pltpu.sync_copy(bnc.at[pl.ds(0, TAIL)],
                                out_hi.at[pl.ds(off, TAIL)])

        for p in range(2):
            chunk_fill()
            plsc.subcore_barrier()

            @pl.when(c == 0)
            def _():
                run(xq[2 * p])

            @pl.when(c == 1)
            def _():
                run(xq[2 * p + 1])

            plsc.subcore_barrier()
            chunk_out(oq[2 * p], oq[2 * p + 1])

    return k(*xqs, src_t, dst_t, zerosq)


def _counts_pallas(dst_t, zeros16, ones16):
    HC = NCHUNK // 2
    out_type = [jax.ShapeDtypeStruct((N_PAD, 16), jnp.float32)] * 2
    scratch = [
        pltpu.VMEM((NCHUNK, EC), jnp.int32),
        pltpu.VMEM((EC, 16), jnp.float32),
        pltpu.VMEM((EC, 16), jnp.float32),
        pltpu.VMEM_SHARED((N_PAD, 16), jnp.float32),
        pltpu.SemaphoreType.DMA,
    ]
    mesh = plsc.VectorSubcoreMesh(core_axis_name="c", subcore_axis_name="s")

    @functools.partial(pl.kernel, out_type=out_type, mesh=mesh,
                       scratch_types=scratch,
                       compiler_params=pltpu.CompilerParams(
                           use_tc_tiling_on_sc=False))
    def k(dst_hbm, z16_hbm, ones_hbm, oc0, oc1, dst_v, ones_v, cbnc, cnt_sh,
          ssem):
        c = lax.axis_index("c")
        s = lax.axis_index("s")
        r0 = s * RPT
        NT, TAIL = RPT // EC, RPT % EC
        pltpu.sync_copy(dst_hbm.at[s], dst_v)
        pltpu.sync_copy(ones_hbm, ones_v)
        pltpu.sync_copy(z16_hbm, cbnc)

        def zf(t, carry):
            pltpu.sync_copy(cbnc, cnt_sh.at[pl.ds(r0 + t * EC, EC)])
            return carry
        lax.fori_loop(0, NT, zf, 0)
        pltpu.sync_copy(cbnc.at[pl.ds(0, TAIL)],
                        cnt_sh.at[pl.ds(r0 + NT * EC, TAIL)])
        plsc.subcore_barrier()

        j0 = c * HC

        def body(j, carry):
            pltpu.async_copy(ones_v, cnt_sh.at[dst_v.at[j0 + j]], ssem,
                             add=True)
            return carry
        lax.fori_loop(0, HC, body, 0)

        def drain(j, carry):
            pltpu.make_async_copy(ones_v, cnt_sh.at[dst_v.at[0]],
                                  ssem).wait()
            return carry
        lax.fori_loop(0, HC, drain, 0)
        plsc.subcore_barrier()

        def cob(t, carry):
            off = r0 + t * EC
            pltpu.sync_copy(cnt_sh.at[pl.ds(off, EC)], cbnc)

            @pl.when(c == 0)
            def _():
                pltpu.sync_copy(cbnc, oc0.at[pl.ds(off, EC)])

            @pl.when(c == 1)
            def _():
                pltpu.sync_copy(cbnc, oc1.at[pl.ds(off, EC)])
            return carry
        lax.fori_loop(0, NT, cob, 0)
        off = r0 + NT * EC
        pltpu.sync_copy(cnt_sh.at[pl.ds(off, TAIL)], cbnc.at[pl.ds(0, TAIL)])

        @pl.when(c == 0)
        def _():
            pltpu.sync_copy(cbnc.at[pl.ds(0, TAIL)], oc0.at[pl.ds(off, TAIL)])

        @pl.when(c == 1)
        def _():
            pltpu.sync_copy(cbnc.at[pl.ds(0, TAIL)], oc1.at[pl.ds(off, TAIL)])

    return k(dst_t, zeros16, ones16)


def _gate_body(*refs, final):
    (xq0, xq1, xq2, xq3, sq0, sq1, sq2, sq3, cnta_ref, cntb_ref, tm_ref,
     twx_ref, twm_ref, tb_ref, g_ref, b_ref) = refs[:16]
    rest = refs[16:]
    if final:
        wout_ref, bout_ref = rest[:2]
        rest = rest[2:]
    o_refs = rest
    x = jnp.concatenate([xq0[...], xq1[...], xq2[...], xq3[...]], axis=-1)
    cnt = cnta_ref[...][:, 0:1] + cntb_ref[...][:, 0:1]
    inv = 1.0 / jnp.maximum(cnt, 1.0)
    m = jnp.concatenate([sq0[...], sq1[...], sq2[...], sq3[...]],
                        axis=-1) * inv
    bf = jnp.bfloat16
    z = (jnp.dot(x.astype(bf), twx_ref[...].astype(bf),
                 preferred_element_type=jnp.float32)
         + jnp.dot(m.astype(bf), twm_ref[...].astype(bf),
                   preferred_element_type=jnp.float32)
         + tb_ref[...])
    z = z - jnp.max(z, axis=-1, keepdims=True)
    ez = jnp.exp(z)
    raw = ez / jnp.sum(ez, axis=-1, keepdims=True)
    ii = lax.broadcasted_iota(jnp.int32, (CH, CH), 0)
    jj = lax.broadcasted_iota(jnp.int32, (CH, CH), 1)
    tri = (ii <= jj).astype(jnp.float32)
    raw = jnp.dot(raw, tri, preferred_element_type=jnp.float32)
    tm = tm_ref[...]
    raw = tm + (1.0 - tm) * raw
    rr = lax.broadcasted_iota(jnp.int32, (CH, HID), 0)
    cc = lax.broadcasted_iota(jnp.int32, (CH, HID), 1)
    rep = HID // CH
    exp_m = ((cc >= rr * rep) & (cc < rr * rep + rep)).astype(jnp.float32)
    sig = jnp.dot(raw, exp_m, preferred_element_type=jnp.float32)
    xn = x * sig + m * (1.0 - sig)
    xn = _ln(xn, g_ref[...], b_ref[...])
    for i in range(NQ):
        o_refs[i][...] = xn[:, i * Q:(i + 1) * Q]
    o_refs[NQ][...] = raw
    if final:
        o_refs[NQ + 1][...] = (
            jnp.dot(xn.astype(bf), wout_ref[...].astype(bf),
                    preferred_element_type=jnp.float32) + bout_ref[...])


def _gate(xqs, sqs, cnt, tm, tw, tb, g, b, wout=None, bout=None):
    final = wout is not None
    grid = N // BLK
    full = lambda r, c: pl.BlockSpec((r, c), lambda i: (0, 0))
    row = lambda c: pl.BlockSpec((BLK, c), lambda i: (i, 0))
    in_specs = ([row(Q)] * NQ + [row(Q)] * NQ
                + [row(16), row(16), row(CH), full(HID, CH), full(HID, CH),
                   full(1, CH), full(1, HID), full(1, HID)])
    args = (list(xqs) + list(sqs)
            + [cnt[0], cnt[1], tm, tw[:HID], tw[HID:], tb.reshape(1, CH),
               g.reshape(1, HID), b.reshape(1, HID)])
    out_specs = [row(Q)] * NQ + [row(CH)]
    out_shape = ([jax.ShapeDtypeStruct((N, Q), jnp.float32)] * NQ
                 + [jax.ShapeDtypeStruct((N, CH), jnp.float32)])
    if final:
        in_specs += [full(HID, OUTD), full(1, OUTD)]
        args += [wout, bout.reshape(1, OUTD)]
        out_specs.append(row(OUTD))
        out_shape.append(jax.ShapeDtypeStruct((N, OUTD), jnp.float32))
    return pl.pallas_call(
        functools.partial(_gate_body, final=final),
        grid=(grid,),
        in_specs=in_specs,
        out_specs=out_specs,
        out_shape=out_shape,
    )(*args)


def _user_gather(y, idx):
    nb = idx.shape[0]
    bpw = nb // (NC * NS)
    mesh = plsc.VectorSubcoreMesh(core_axis_name="c", subcore_axis_name="s")

    @functools.partial(
        pl.kernel, mesh=mesh,
        out_type=jax.ShapeDtypeStruct((nb, OUTD), jnp.float32),
        scratch_types=[
            pltpu.VMEM((bpw,), jnp.int32),
            pltpu.VMEM((bpw, OUTD), jnp.float32),
            pltpu.SemaphoreType.DMA,
        ],
    )
    def k(y_hbm, idx_hbm, out_hbm, idx_v, rows_v, sem):
        wid = lax.axis_index("s") * NC + lax.axis_index("c")
        base = wid * bpw
        pltpu.sync_copy(idx_hbm.at[pl.ds(base, bpw)], idx_v)
        pltpu.async_copy(y_hbm.at[idx_v], rows_v, sem).wait()
        pltpu.sync_copy(rows_v, out_hbm.at[pl.ds(base, bpw)])

    return k(y, idx)


def kernel(user_idx, edge_index, x_table, Win0, bin0, Win1, bin1, ln0_g,
           ln0_b, ln1_g, ln1_b, tm0_W, tm0_b, tm1_W, tm1_b, tmn0_g, tmn0_b,
           tmn1_g, tmn1_b, Wout, bout):
    f32 = jnp.float32
    user_idx = user_idx.astype(jnp.int32)
    src = edge_index[0].astype(jnp.int32)
    dst = edge_index[1].astype(jnp.int32)
    npad = E_PAD - E
    src_t = jnp.concatenate([src, jnp.zeros((npad,), jnp.int32)])
    src_t = src_t.reshape(NS, NCHUNK, EC)
    dst_t = jnp.concatenate([dst, jnp.full((npad,), N, jnp.int32)])
    dst_t = dst_t.reshape(NS, NCHUNK, EC)
    zerosq = jnp.zeros((EC, Q), f32)
    zeros16 = jnp.zeros((EC, 16), f32)
    ones16 = jnp.ones((EC, 16), f32)

    cnt = _counts_pallas(dst_t, zeros16, ones16)
    xqs = _mlp(x_table, Win0, bin0, Win1, bin1, ln0_g, ln0_b, ln1_g, ln1_b)

    sqs = _segsum_pallas(xqs, src_t, dst_t, zerosq)
    tm0 = jnp.zeros((N, CH), f32)
    *xg1, tm1 = _gate(xqs, sqs, cnt, tm0, tm0_W, tm0_b, tmn0_g, tmn0_b)

    sqs2 = _segsum_pallas(xg1, src_t, dst_t, zerosq)
    outs = _gate(xg1, sqs2, cnt, tm1, tm1_W, tm1_b, tmn1_g, tmn1_b,
                 wout=Wout, bout=bout)
    y = outs[-1]

    user_embedding = _user_gather(y, user_idx)
    item_embedding = y[USERS:]
    return (user_embedding, item_embedding)


# speedup vs baseline: 3.5817x; 1.0043x over previous
import functools

import jax
import jax.numpy as jnp
from jax import lax
from jax.experimental import pallas as pl
from jax.experimental.pallas import tpu as pltpu
from jax.experimental.pallas import tpu_sc as plsc

N = 10000
USERS = 2000
HID = 256
Q = 64
NQ = 4
CH = 64
OUTD = 128
E = 160000

NC = 2
NS = 16
EC = 128
NCHUNK = 80
E_PAD = NS * NCHUNK * EC
N_PAD = 10112
RPT = N_PAD // NS

BLK = 1000


def _ln(x, g, b, eps=1e-5):
    mu = jnp.mean(x, axis=-1, keepdims=True)
    var = jnp.mean((x - mu) ** 2, axis=-1, keepdims=True)
    return (x - mu) / jnp.sqrt(var + eps) * g + b


def _mlp_body(x_ref, w0_ref, b0_ref, w1_ref, b1_ref, g0_ref, bb0_ref,
              g1_ref, bb1_ref, *o_refs):
    bf = jnp.bfloat16
    x = x_ref[...]
    h = jnp.maximum(
        jnp.dot(x.astype(bf), w0_ref[...].astype(bf),
                preferred_element_type=jnp.float32) + b0_ref[...], 0.0)
    h = _ln(h, g0_ref[...], bb0_ref[...])
    h = jnp.maximum(
        jnp.dot(h.astype(bf), w1_ref[...].astype(bf),
                preferred_element_type=jnp.float32) + b1_ref[...], 0.0)
    h = _ln(h, g1_ref[...], bb1_ref[...])
    for i in range(NQ):
        o_refs[i][...] = h[:, i * Q:(i + 1) * Q]


def _mlp(x, w0, b0, w1, b1, g0, bb0, g1, bb1):
    grid = N // BLK
    full = lambda r, c: pl.BlockSpec((r, c), lambda i: (0, 0))
    return pl.pallas_call(
        _mlp_body,
        grid=(grid,),
        in_specs=[
            pl.BlockSpec((BLK, HID), lambda i: (i, 0)),
            full(HID, HID), full(1, HID), full(HID, HID), full(1, HID),
            full(1, HID), full(1, HID), full(1, HID), full(1, HID),
        ],
        out_specs=[pl.BlockSpec((BLK, Q), lambda i: (i, 0))] * NQ,
        out_shape=[jax.ShapeDtypeStruct((N, Q), jnp.float32)] * NQ,
    )(x, w0, b0.reshape(1, HID), w1, b1.reshape(1, HID),
      g0.reshape(1, HID), bb0.reshape(1, HID), g1.reshape(1, HID),
      bb1.reshape(1, HID))


def _segsum_pallas(xqs, src_t, dst_t, zerosq):
    out_type = [jax.ShapeDtypeStruct((N_PAD, Q), jnp.float32)] * NQ
    G = 5
    scratch = [
        pltpu.VMEM((NCHUNK, EC), jnp.int32),
        pltpu.VMEM((NCHUNK, EC), jnp.int32),
        pltpu.VMEM((EC, Q), jnp.float32),
        pltpu.VMEM_SHARED((N_PAD, Q), jnp.float32),
    ] + [pltpu.VMEM((EC, Q), jnp.float32)] * G \
      + [pltpu.SemaphoreType.DMA] * (2 * G)
    mesh = plsc.VectorSubcoreMesh(core_axis_name="c", subcore_axis_name="s")

    @functools.partial(pl.kernel, out_type=out_type, mesh=mesh,
                       scratch_types=scratch,
                       compiler_params=pltpu.CompilerParams(
                           use_tc_tiling_on_sc=False))
    def k(xq0, xq1, xq2, xq3, src_hbm, dst_hbm, zq_hbm,
          oq0, oq1, oq2, oq3, *rest):
        src_v, dst_v, bnc, acc_sh = rest[:4]
        gb = rest[4:4 + G]
        gsem = rest[4 + G:4 + 2 * G]
        ssems = rest[4 + 2 * G:4 + 3 * G]
        xq = (xq0, xq1, xq2, xq3)
        oq = (oq0, oq1, oq2, oq3)
        c = lax.axis_index("c")
        s = lax.axis_index("s")
        r0 = s * RPT
        NT, TAIL = RPT // EC, RPT % EC

        pltpu.sync_copy(src_hbm.at[s], src_v)
        pltpu.sync_copy(dst_hbm.at[s], dst_v)

        def chunk_fill():
            pltpu.sync_copy(zq_hbm, bnc)

            def zf(t, carry):
                pltpu.sync_copy(bnc, acc_sh.at[pl.ds(r0 + t * EC, EC)])
                return carry
            lax.fori_loop(0, NT, zf, 0)
            pltpu.sync_copy(bnc.at[pl.ds(0, TAIL)],
                            acc_sh.at[pl.ds(r0 + NT * EC, TAIL)])

        def run(x_hbm):
            def wait_scatter(b):
                pltpu.make_async_copy(
                    gb[b], acc_sh.at[dst_v.at[0]], ssems[b]).wait()

            def body(it, carry):
                j0 = it * G
                gds = []
                for b in range(G):
                    @pl.when(it > 0)
                    def _(b=b):
                        wait_scatter(b)
                    gds.append(pltpu.async_copy(
                        x_hbm.at[src_v.at[j0 + b]], gb[b], gsem[b]))
                for b in range(G):
                    gds[b].wait()
                    pltpu.async_copy(
                        gb[b], acc_sh.at[dst_v.at[j0 + b]], ssems[b],
                        add=True)
                return carry
            lax.fori_loop(0, NCHUNK // G, body, 0)
            for b in range(G):
                wait_scatter(b)

        def chunk_out(out_lo, out_hi):
            def cob(t, carry):
                off = r0 + t * EC
                pltpu.sync_copy(acc_sh.at[pl.ds(off, EC)], bnc)

                @pl.when(c == 0)
                def _():
                    pltpu.sync_copy(bnc, out_lo.at[pl.ds(off, EC)])

                @pl.when(c == 1)
                def _():
                    pltpu.sync_copy(bnc, out_hi.at[pl.ds(off, EC)])
                return carry
            lax.fori_loop(0, NT, cob, 0)
            off = r0 + NT * EC
            pltpu.sync_copy(acc_sh.at[pl.ds(off, TAIL)],
                            bnc.at[pl.ds(0, TAIL)])

            @pl.when(c == 0)
            def _():
                pltpu.sync_copy(bnc.at[pl.ds(0, TAIL)],
                                out_lo.at[pl.ds(off, TAIL)])

            @pl.when(c == 1)
            def _():
                pltpu.sync_copy(bnc.at[pl.ds(0, TAIL)],
                                out_hi.at[pl.ds(off, TAIL)])

        for p in range(2):
            chunk_fill()
            plsc.subcore_barrier()

            @pl.when(c == 0)
            def _():
                run(xq[2 * p])

            @pl.when(c == 1)
            def _():
                run(xq[2 * p + 1])

            plsc.subcore_barrier()
            chunk_out(oq[2 * p], oq[2 * p + 1])

    return k(*xqs, src_t, dst_t, zerosq)


def _counts_pallas(dst_t, zeros16, ones16):
    HC = NCHUNK // 2
    out_type = [jax.ShapeDtypeStruct((N_PAD, 16), jnp.float32)] * 2
    scratch = [
        pltpu.VMEM((NCHUNK, EC), jnp.int32),
        pltpu.VMEM((EC, 16), jnp.float32),
        pltpu.VMEM((EC, 16), jnp.float32),
        pltpu.VMEM_SHARED((N_PAD, 16), jnp.float32),
        pltpu.SemaphoreType.DMA,
    ]
    mesh = plsc.VectorSubcoreMesh(core_axis_name="c", subcore_axis_name="s")

    @functools.partial(pl.kernel, out_type=out_type, mesh=mesh,
                       scratch_types=scratch,
                       compiler_params=pltpu.CompilerParams(
                           use_tc_tiling_on_sc=False))
    def k(dst_hbm, z16_hbm, ones_hbm, oc0, oc1, dst_v, ones_v, cbnc, cnt_sh,
          ssem):
        c = lax.axis_index("c")
        s = lax.axis_index("s")
        r0 = s * RPT
        NT, TAIL = RPT // EC, RPT % EC
        pltpu.sync_copy(dst_hbm.at[s], dst_v)
        pltpu.sync_copy(ones_hbm, ones_v)
        pltpu.sync_copy(z16_hbm, cbnc)

        def zf(t, carry):
            pltpu.sync_copy(cbnc, cnt_sh.at[pl.ds(r0 + t * EC, EC)])
            return carry
        lax.fori_loop(0, NT, zf, 0)
        pltpu.sync_copy(cbnc.at[pl.ds(0, TAIL)],
                        cnt_sh.at[pl.ds(r0 + NT * EC, TAIL)])
        plsc.subcore_barrier()

        j0 = c * HC

        def body(j, carry):
            pltpu.async_copy(ones_v, cnt_sh.at[dst_v.at[j0 + j]], ssem,
                             add=True)
            return carry
        lax.fori_loop(0, HC, body, 0)

        def drain(j, carry):
            pltpu.make_async_copy(ones_v, cnt_sh.at[dst_v.at[0]],
                                  ssem).wait()
            return carry
        lax.fori_loop(0, HC, drain, 0)
        plsc.subcore_barrier()

        def cob(t, carry):
            off = r0 + t * EC
            pltpu.sync_copy(cnt_sh.at[pl.ds(off, EC)], cbnc)

            @pl.when(c == 0)
            def _():
                pltpu.sync_copy(cbnc, oc0.at[pl.ds(off, EC)])

            @pl.when(c == 1)
            def _():
                pltpu.sync_copy(cbnc, oc1.at[pl.ds(off, EC)])
            return carry
        lax.fori_loop(0, NT, cob, 0)
        off = r0 + NT * EC
        pltpu.sync_copy(cnt_sh.at[pl.ds(off, TAIL)], cbnc.at[pl.ds(0, TAIL)])

        @pl.when(c == 0)
        def _():
            pltpu.sync_copy(cbnc.at[pl.ds(0, TAIL)], oc0.at[pl.ds(off, TAIL)])

        @pl.when(c == 1)
        def _():
            pltpu.sync_copy(cbnc.at[pl.ds(0, TAIL)], oc1.at[pl.ds(off, TAIL)])

    return k(dst_t, zeros16, ones16)


def _gate_body(*refs, final, has_tm):
    (xq0, xq1, xq2, xq3, sq0, sq1, sq2, sq3, cnta_ref,
     cntb_ref) = refs[:10]
    rest = refs[10:]
    tm_ref = None
    if has_tm:
        tm_ref = rest[0]
        rest = rest[1:]
    twx_ref, twm_ref, tb_ref, g_ref, b_ref = rest[:5]
    rest = rest[5:]
    if final:
        wout_ref, bout_ref = rest[:2]
        rest = rest[2:]
    o_refs = rest
    x = jnp.concatenate([xq0[...], xq1[...], xq2[...], xq3[...]], axis=-1)
    cnt = cnta_ref[...][:, 0:1] + cntb_ref[...][:, 0:1]
    inv = 1.0 / jnp.maximum(cnt, 1.0)
    m = jnp.concatenate([sq0[...], sq1[...], sq2[...], sq3[...]],
                        axis=-1) * inv
    bf = jnp.bfloat16
    z = (jnp.dot(x.astype(bf), twx_ref[...].astype(bf),
                 preferred_element_type=jnp.float32)
         + jnp.dot(m.astype(bf), twm_ref[...].astype(bf),
                   preferred_element_type=jnp.float32)
         + tb_ref[...])
    z = z - jnp.max(z, axis=-1, keepdims=True)
    ez = jnp.exp(z)
    raw = ez / jnp.sum(ez, axis=-1, keepdims=True)
    ii = lax.broadcasted_iota(jnp.int32, (CH, CH), 0)
    jj = lax.broadcasted_iota(jnp.int32, (CH, CH), 1)
    tri = (ii <= jj).astype(jnp.float32)
    raw = jnp.dot(raw, tri, preferred_element_type=jnp.float32)
    if has_tm:
        tm = tm_ref[...]
        raw = tm + (1.0 - tm) * raw
    rr = lax.broadcasted_iota(jnp.int32, (CH, HID), 0)
    cc = lax.broadcasted_iota(jnp.int32, (CH, HID), 1)
    rep = HID // CH
    exp_m = ((cc >= rr * rep) & (cc < rr * rep + rep)).astype(jnp.float32)
    sig = jnp.dot(raw, exp_m, preferred_element_type=jnp.float32)
    xn = x * sig + m * (1.0 - sig)
    xn = _ln(xn, g_ref[...], b_ref[...])
    for i in range(NQ):
        o_refs[i][...] = xn[:, i * Q:(i + 1) * Q]
    o_refs[NQ][...] = raw
    if final:
        y = (jnp.dot(xn.astype(bf), wout_ref[...].astype(bf),
                     preferred_element_type=jnp.float32) + bout_ref[...])
        i = pl.program_id(0)
        nh = USERS // BLK

        @pl.when(i < nh)
        def _():
            o_refs[NQ + 1][...] = y

        @pl.when(i >= nh)
        def _():
            o_refs[NQ + 2][...] = y


def _gate(xqs, sqs, cnt, tm, tw, tb, g, b, wout=None, bout=None):
    final = wout is not None
    grid = N // BLK
    full = lambda r, c: pl.BlockSpec((r, c), lambda i: (0, 0))
    row = lambda c: pl.BlockSpec((BLK, c), lambda i: (i, 0))
    has_tm = tm is not None
    in_specs = ([row(Q)] * NQ + [row(Q)] * NQ + [row(16), row(16)]
                + ([row(CH)] if has_tm else [])
                + [full(HID, CH), full(HID, CH),
                   full(1, CH), full(1, HID), full(1, HID)])
    args = (list(xqs) + list(sqs) + [cnt[0], cnt[1]]
            + ([tm] if has_tm else [])
            + [tw[:HID], tw[HID:], tb.reshape(1, CH),
               g.reshape(1, HID), b.reshape(1, HID)])
    out_specs = [row(Q)] * NQ + [row(CH)]
    out_shape = ([jax.ShapeDtypeStruct((N, Q), jnp.float32)] * NQ
                 + [jax.ShapeDtypeStruct((N, CH), jnp.float32)])
    if final:
        nh = USERS // BLK
        in_specs += [full(HID, OUTD), full(1, OUTD)]
        args += [wout, bout.reshape(1, OUTD)]
        out_specs.append(pl.BlockSpec(
            (BLK, OUTD), lambda i: (jnp.minimum(i, nh - 1), 0)))
        out_specs.append(pl.BlockSpec(
            (BLK, OUTD), lambda i: (jnp.maximum(i - nh, 0), 0)))
        out_shape.append(jax.ShapeDtypeStruct((USERS, OUTD), jnp.float32))
        out_shape.append(jax.ShapeDtypeStruct((N - USERS, OUTD),
                                              jnp.float32))
    return pl.pallas_call(
        functools.partial(_gate_body, final=final, has_tm=has_tm),
        grid=(grid,),
        in_specs=in_specs,
        out_specs=out_specs,
        out_shape=out_shape,
    )(*args)


def _user_gather(y, idx):
    nb = idx.shape[0]
    bpw = nb // (NC * NS)
    mesh = plsc.VectorSubcoreMesh(core_axis_name="c", subcore_axis_name="s")

    @functools.partial(
        pl.kernel, mesh=mesh,
        out_type=jax.ShapeDtypeStruct((nb, OUTD), jnp.float32),
        scratch_types=[
            pltpu.VMEM((bpw,), jnp.int32),
            pltpu.VMEM((bpw, OUTD), jnp.float32),
            pltpu.SemaphoreType.DMA,
        ],
    )
    def k(y_hbm, idx_hbm, out_hbm, idx_v, rows_v, sem):
        wid = lax.axis_index("s") * NC + lax.axis_index("c")
        base = wid * bpw
        pltpu.sync_copy(idx_hbm.at[pl.ds(base, bpw)], idx_v)
        pltpu.async_copy(y_hbm.at[idx_v], rows_v, sem).wait()
        pltpu.sync_copy(rows_v, out_hbm.at[pl.ds(base, bpw)])

    return k(y, idx)


def kernel(user_idx, edge_index, x_table, Win0, bin0, Win1, bin1, ln0_g,
           ln0_b, ln1_g, ln1_b, tm0_W, tm0_b, tm1_W, tm1_b, tmn0_g, tmn0_b,
           tmn1_g, tmn1_b, Wout, bout):
    f32 = jnp.float32
    user_idx = user_idx.astype(jnp.int32)
    src = edge_index[0].astype(jnp.int32)
    dst = edge_index[1].astype(jnp.int32)
    npad = E_PAD - E
    src_t = jnp.concatenate([src, jnp.zeros((npad,), jnp.int32)])
    src_t = src_t.reshape(NS, NCHUNK, EC)
    dst_t = jnp.concatenate([dst, jnp.full((npad,), N, jnp.int32)])
    dst_t = dst_t.reshape(NS, NCHUNK, EC)
    zerosq = jnp.zeros((EC, Q), f32)
    zeros16 = jnp.zeros((EC, 16), f32)
    ones16 = jnp.ones((EC, 16), f32)

    cnt = _counts_pallas(dst_t, zeros16, ones16)
    xqs = _mlp(x_table, Win0, bin0, Win1, bin1, ln0_g, ln0_b, ln1_g, ln1_b)

    sqs = _segsum_pallas(xqs, src_t, dst_t, zerosq)
    *xg1, tm1 = _gate(xqs, sqs, cnt, None, tm0_W, tm0_b, tmn0_g, tmn0_b)

    sqs2 = _segsum_pallas(xg1, src_t, dst_t, zerosq)
    outs = _gate(xg1, sqs2, cnt, tm1, tm1_W, tm1_b, tmn1_g, tmn1_b,
                 wout=Wout, bout=bout)
    y_head, y_items = outs[-2], outs[-1]

    user_embedding = _user_gather(y_head, user_idx)
    return (user_embedding, y_items)


# speedup vs baseline: 3.6356x; 1.0151x over previous
import functools

import jax
import jax.numpy as jnp
from jax import lax
from jax.experimental import pallas as pl
from jax.experimental.pallas import tpu as pltpu
from jax.experimental.pallas import tpu_sc as plsc

N = 10000
USERS = 2000
HID = 256
Q = 64
NQ = 4
CH = 64
OUTD = 128
E = 160000

NC = 2
NS = 16
EC = 128
NCHUNK = 80
E_PAD = NS * NCHUNK * EC
N_PAD = 10112
RPT = N_PAD // NS

BLK = 2000


def _ln(x, g, b, eps=1e-5):
    mu = jnp.mean(x, axis=-1, keepdims=True)
    var = jnp.mean((x - mu) ** 2, axis=-1, keepdims=True)
    return (x - mu) / jnp.sqrt(var + eps) * g + b


def _mlp_body(x_ref, w0_ref, b0_ref, w1_ref, b1_ref, g0_ref, bb0_ref,
              g1_ref, bb1_ref, *o_refs):
    bf = jnp.bfloat16
    x = x_ref[...]
    h = jnp.maximum(
        jnp.dot(x.astype(bf), w0_ref[...].astype(bf),
                preferred_element_type=jnp.float32) + b0_ref[...], 0.0)
    h = _ln(h, g0_ref[...], bb0_ref[...])
    h = jnp.maximum(
        jnp.dot(h.astype(bf), w1_ref[...].astype(bf),
                preferred_element_type=jnp.float32) + b1_ref[...], 0.0)
    h = _ln(h, g1_ref[...], bb1_ref[...])
    for i in range(NQ):
        o_refs[i][...] = h[:, i * Q:(i + 1) * Q]


def _mlp(x, w0, b0, w1, b1, g0, bb0, g1, bb1):
    grid = N // BLK
    full = lambda r, c: pl.BlockSpec((r, c), lambda i: (0, 0))
    return pl.pallas_call(
        _mlp_body,
        grid=(grid,),
        in_specs=[
            pl.BlockSpec((BLK, HID), lambda i: (i, 0)),
            full(HID, HID), full(1, HID), full(HID, HID), full(1, HID),
            full(1, HID), full(1, HID), full(1, HID), full(1, HID),
        ],
        out_specs=[pl.BlockSpec((BLK, Q), lambda i: (i, 0))] * NQ,
        out_shape=[jax.ShapeDtypeStruct((N, Q), jnp.float32)] * NQ,
    )(x, w0, b0.reshape(1, HID), w1, b1.reshape(1, HID),
      g0.reshape(1, HID), bb0.reshape(1, HID), g1.reshape(1, HID),
      bb1.reshape(1, HID))


def _segsum_pallas(xqs, src_t, dst_t, zerosq):
    out_type = [jax.ShapeDtypeStruct((N_PAD, Q), jnp.float32)] * NQ
    G = 5
    scratch = [
        pltpu.VMEM((NCHUNK, EC), jnp.int32),
        pltpu.VMEM((NCHUNK, EC), jnp.int32),
        pltpu.VMEM((EC, Q), jnp.float32),
        pltpu.VMEM_SHARED((N_PAD, Q), jnp.float32),
    ] + [pltpu.VMEM((EC, Q), jnp.float32)] * G \
      + [pltpu.SemaphoreType.DMA] * (2 * G)
    mesh = plsc.VectorSubcoreMesh(core_axis_name="c", subcore_axis_name="s")

    @functools.partial(pl.kernel, out_type=out_type, mesh=mesh,
                       scratch_types=scratch,
                       compiler_params=pltpu.CompilerParams(
                           use_tc_tiling_on_sc=False))
    def k(xq0, xq1, xq2, xq3, src_hbm, dst_hbm, zq_hbm,
          oq0, oq1, oq2, oq3, *rest):
        src_v, dst_v, bnc, acc_sh = rest[:4]
        gb = rest[4:4 + G]
        gsem = rest[4 + G:4 + 2 * G]
        ssems = rest[4 + 2 * G:4 + 3 * G]
        xq = (xq0, xq1, xq2, xq3)
        oq = (oq0, oq1, oq2, oq3)
        c = lax.axis_index("c")
        s = lax.axis_index("s")
        r0 = s * RPT
        NT, TAIL = RPT // EC, RPT % EC

        pltpu.sync_copy(src_hbm.at[s], src_v)
        pltpu.sync_copy(dst_hbm.at[s], dst_v)

        def chunk_fill():
            pltpu.sync_copy(zq_hbm, bnc)

            def zf(t, carry):
                pltpu.sync_copy(bnc, acc_sh.at[pl.ds(r0 + t * EC, EC)])
                return carry
            lax.fori_loop(0, NT, zf, 0)
            pltpu.sync_copy(bnc.at[pl.ds(0, TAIL)],
                            acc_sh.at[pl.ds(r0 + NT * EC, TAIL)])

        def run(x_hbm):
            def wait_scatter(b):
                pltpu.make_async_copy(
                    gb[b], acc_sh.at[dst_v.at[0]], ssems[b]).wait()

            def body(it, carry):
                j0 = it * G
                gds = []
                for b in range(G):
                    @pl.when(it > 0)
                    def _(b=b):
                        wait_scatter(b)
                    gds.append(pltpu.async_copy(
                        x_hbm.at[src_v.at[j0 + b]], gb[b], gsem[b]))
                for b in range(G):
                    gds[b].wait()
                    pltpu.async_copy(
                        gb[b], acc_sh.at[dst_v.at[j0 + b]], ssems[b],
                        add=True)
                return carry
            lax.fori_loop(0, NCHUNK // G, body, 0)
            for b in range(G):
                wait_scatter(b)

        def chunk_out(out_lo, out_hi):
            def cob(t, carry):
                off = r0 + t * EC
                pltpu.sync_copy(acc_sh.at[pl.ds(off, EC)], bnc)

                @pl.when(c == 0)
                def _():
                    pltpu.sync_copy(bnc, out_lo.at[pl.ds(off, EC)])

                @pl.when(c == 1)
                def _():
                    pltpu.sync_copy(bnc, out_hi.at[pl.ds(off, EC)])
                return carry
            lax.fori_loop(0, NT, cob, 0)
            off = r0 + NT * EC
            pltpu.sync_copy(acc_sh.at[pl.ds(off, TAIL)],
                            bnc.at[pl.ds(0, TAIL)])

            @pl.when(c == 0)
            def _():
                pltpu.sync_copy(bnc.at[pl.ds(0, TAIL)],
                                out_lo.at[pl.ds(off, TAIL)])

            @pl.when(c == 1)
            def _():
                pltpu.sync_copy(bnc.at[pl.ds(0, TAIL)],
                                out_hi.at[pl.ds(off, TAIL)])

        for p in range(2):
            chunk_fill()
            plsc.subcore_barrier()

            @pl.when(c == 0)
            def _():
                run(xq[2 * p])

            @pl.when(c == 1)
            def _():
                run(xq[2 * p + 1])

            plsc.subcore_barrier()
            chunk_out(oq[2 * p], oq[2 * p + 1])

    return k(*xqs, src_t, dst_t, zerosq)


def _counts_pallas(dst_t, zeros16, ones16):
    HC = NCHUNK // 2
    out_type = [jax.ShapeDtypeStruct((N_PAD, 16), jnp.float32)] * 2
    scratch = [
        pltpu.VMEM((NCHUNK, EC), jnp.int32),
        pltpu.VMEM((EC, 16), jnp.float32),
        pltpu.VMEM((EC, 16), jnp.float32),
        pltpu.VMEM_SHARED((N_PAD, 16), jnp.float32),
        pltpu.SemaphoreType.DMA,
    ]
    mesh = plsc.VectorSubcoreMesh(core_axis_name="c", subcore_axis_name="s")

    @functools.partial(pl.kernel, out_type=out_type, mesh=mesh,
                       scratch_types=scratch,
                       compiler_params=pltpu.CompilerParams(
                           use_tc_tiling_on_sc=False))
    def k(dst_hbm, z16_hbm, ones_hbm, oc0, oc1, dst_v, ones_v, cbnc, cnt_sh,
          ssem):
        c = lax.axis_index("c")
        s = lax.axis_index("s")
        r0 = s * RPT
        NT, TAIL = RPT // EC, RPT % EC
        pltpu.sync_copy(dst_hbm.at[s], dst_v)
        pltpu.sync_copy(ones_hbm, ones_v)
        pltpu.sync_copy(z16_hbm, cbnc)

        def zf(t, carry):
            pltpu.sync_copy(cbnc, cnt_sh.at[pl.ds(r0 + t * EC, EC)])
            return carry
        lax.fori_loop(0, NT, zf, 0)
        pltpu.sync_copy(cbnc.at[pl.ds(0, TAIL)],
                        cnt_sh.at[pl.ds(r0 + NT * EC, TAIL)])
        plsc.subcore_barrier()

        j0 = c * HC

        def body(j, carry):
            pltpu.async_copy(ones_v, cnt_sh.at[dst_v.at[j0 + j]], ssem,
                             add=True)
            return carry
        lax.fori_loop(0, HC, body, 0)

        def drain(j, carry):
            pltpu.make_async_copy(ones_v, cnt_sh.at[dst_v.at[0]],
                                  ssem).wait()
            return carry
        lax.fori_loop(0, HC, drain, 0)
        plsc.subcore_barrier()

        def cob(t, carry):
            off = r0 + t * EC
            pltpu.sync_copy(cnt_sh.at[pl.ds(off, EC)], cbnc)

            @pl.when(c == 0)
            def _():
                pltpu.sync_copy(cbnc, oc0.at[pl.ds(off, EC)])

            @pl.when(c == 1)
            def _():
                pltpu.sync_copy(cbnc, oc1.at[pl.ds(off, EC)])
            return carry
        lax.fori_loop(0, NT, cob, 0)
        off = r0 + NT * EC
        pltpu.sync_copy(cnt_sh.at[pl.ds(off, TAIL)], cbnc.at[pl.ds(0, TAIL)])

        @pl.when(c == 0)
        def _():
            pltpu.sync_copy(cbnc.at[pl.ds(0, TAIL)], oc0.at[pl.ds(off, TAIL)])

        @pl.when(c == 1)
        def _():
            pltpu.sync_copy(cbnc.at[pl.ds(0, TAIL)], oc1.at[pl.ds(off, TAIL)])

    return k(dst_t, zeros16, ones16)


def _gate_body(*refs, final, has_tm):
    (xq0, xq1, xq2, xq3, sq0, sq1, sq2, sq3, cnta_ref,
     cntb_ref) = refs[:10]
    rest = refs[10:]
    tm_ref = None
    if has_tm:
        tm_ref = rest[0]
        rest = rest[1:]
    twx_ref, twm_ref, tb_ref, g_ref, b_ref = rest[:5]
    rest = rest[5:]
    if final:
        wout_ref, bout_ref = rest[:2]
        rest = rest[2:]
    o_refs = rest
    x = jnp.concatenate([xq0[...], xq1[...], xq2[...], xq3[...]], axis=-1)
    cnt = cnta_ref[...][:, 0:1] + cntb_ref[...][:, 0:1]
    inv = 1.0 / jnp.maximum(cnt, 1.0)
    m = jnp.concatenate([sq0[...], sq1[...], sq2[...], sq3[...]],
                        axis=-1) * inv
    bf = jnp.bfloat16
    z = (jnp.dot(x.astype(bf), twx_ref[...].astype(bf),
                 preferred_element_type=jnp.float32)
         + jnp.dot(m.astype(bf), twm_ref[...].astype(bf),
                   preferred_element_type=jnp.float32)
         + tb_ref[...])
    z = z - jnp.max(z, axis=-1, keepdims=True)
    ez = jnp.exp(z)
    raw = ez / jnp.sum(ez, axis=-1, keepdims=True)
    ii = lax.broadcasted_iota(jnp.int32, (CH, CH), 0)
    jj = lax.broadcasted_iota(jnp.int32, (CH, CH), 1)
    tri = (ii <= jj).astype(jnp.float32)
    raw = jnp.dot(raw, tri, preferred_element_type=jnp.float32)
    if has_tm:
        tm = tm_ref[...]
        raw = tm + (1.0 - tm) * raw
    rr = lax.broadcasted_iota(jnp.int32, (CH, HID), 0)
    cc = lax.broadcasted_iota(jnp.int32, (CH, HID), 1)
    rep = HID // CH
    exp_m = ((cc >= rr * rep) & (cc < rr * rep + rep)).astype(jnp.float32)
    sig = jnp.dot(raw, exp_m, preferred_element_type=jnp.float32)
    xn = x * sig + m * (1.0 - sig)
    xn = _ln(xn, g_ref[...], b_ref[...])
    for i in range(NQ):
        o_refs[i][...] = xn[:, i * Q:(i + 1) * Q]
    o_refs[NQ][...] = raw
    if final:
        y = (jnp.dot(xn.astype(bf), wout_ref[...].astype(bf),
                     preferred_element_type=jnp.float32) + bout_ref[...])
        i = pl.program_id(0)
        nh = USERS // BLK

        @pl.when(i < nh)
        def _():
            o_refs[NQ + 1][...] = y

        @pl.when(i >= nh)
        def _():
            o_refs[NQ + 2][...] = y


def _gate(xqs, sqs, cnt, tm, tw, tb, g, b, wout=None, bout=None):
    final = wout is not None
    grid = N // BLK
    full = lambda r, c: pl.BlockSpec((r, c), lambda i: (0, 0))
    row = lambda c: pl.BlockSpec((BLK, c), lambda i: (i, 0))
    has_tm = tm is not None
    in_specs = ([row(Q)] * NQ + [row(Q)] * NQ + [row(16), row(16)]
                + ([row(CH)] if has_tm else [])
                + [full(HID, CH), full(HID, CH),
                   full(1, CH), full(1, HID), full(1, HID)])
    args = (list(xqs) + list(sqs) + [cnt[0], cnt[1]]
            + ([tm] if has_tm else [])
            + [tw[:HID], tw[HID:], tb.reshape(1, CH),
               g.reshape(1, HID), b.reshape(1, HID)])
    out_specs = [row(Q)] * NQ + [row(CH)]
    out_shape = ([jax.ShapeDtypeStruct((N, Q), jnp.float32)] * NQ
                 + [jax.ShapeDtypeStruct((N, CH), jnp.float32)])
    if final:
        nh = USERS // BLK
        in_specs += [full(HID, OUTD), full(1, OUTD)]
        args += [wout, bout.reshape(1, OUTD)]
        out_specs.append(pl.BlockSpec(
            (BLK, OUTD), lambda i: (jnp.minimum(i, nh - 1), 0)))
        out_specs.append(pl.BlockSpec(
            (BLK, OUTD), lambda i: (jnp.maximum(i - nh, 0), 0)))
        out_shape.append(jax.ShapeDtypeStruct((USERS, OUTD), jnp.float32))
        out_shape.append(jax.ShapeDtypeStruct((N - USERS, OUTD),
                                              jnp.float32))
    return pl.pallas_call(
        functools.partial(_gate_body, final=final, has_tm=has_tm),
        grid=(grid,),
        in_specs=in_specs,
        out_specs=out_specs,
        out_shape=out_shape,
    )(*args)


def _user_gather(y, idx):
    nb = idx.shape[0]
    bpw = nb // (NC * NS)
    mesh = plsc.VectorSubcoreMesh(core_axis_name="c", subcore_axis_name="s")

    @functools.partial(
        pl.kernel, mesh=mesh,
        out_type=jax.ShapeDtypeStruct((nb, OUTD), jnp.float32),
        scratch_types=[
            pltpu.VMEM((bpw,), jnp.int32),
            pltpu.VMEM((bpw, OUTD), jnp.float32),
            pltpu.SemaphoreType.DMA,
        ],
    )
    def k(y_hbm, idx_hbm, out_hbm, idx_v, rows_v, sem):
        wid = lax.axis_index("s") * NC + lax.axis_index("c")
        base = wid * bpw
        pltpu.sync_copy(idx_hbm.at[pl.ds(base, bpw)], idx_v)
        pltpu.async_copy(y_hbm.at[idx_v], rows_v, sem).wait()
        pltpu.sync_copy(rows_v, out_hbm.at[pl.ds(base, bpw)])

    return k(y, idx)


def kernel(user_idx, edge_index, x_table, Win0, bin0, Win1, bin1, ln0_g,
           ln0_b, ln1_g, ln1_b, tm0_W, tm0_b, tm1_W, tm1_b, tmn0_g, tmn0_b,
           tmn1_g, tmn1_b, Wout, bout):
    f32 = jnp.float32
    user_idx = user_idx.astype(jnp.int32)
    src = edge_index[0].astype(jnp.int32)
    dst = edge_index[1].astype(jnp.int32)
    npad = E_PAD - E
    src_t = jnp.concatenate([src, jnp.zeros((npad,), jnp.int32)])
    src_t = src_t.reshape(NS, NCHUNK, EC)
    dst_t = jnp.concatenate([dst, jnp.full((npad,), N, jnp.int32)])
    dst_t = dst_t.reshape(NS, NCHUNK, EC)
    zerosq = jnp.zeros((EC, Q), f32)
    zeros16 = jnp.zeros((EC, 16), f32)
    ones16 = jnp.ones((EC, 16), f32)

    cnt = _counts_pallas(dst_t, zeros16, ones16)
    xqs = _mlp(x_table, Win0, bin0, Win1, bin1, ln0_g, ln0_b, ln1_g, ln1_b)

    sqs = _segsum_pallas(xqs, src_t, dst_t, zerosq)
    *xg1, tm1 = _gate(xqs, sqs, cnt, None, tm0_W, tm0_b, tmn0_g, tmn0_b)

    sqs2 = _segsum_pallas(xg1, src_t, dst_t, zerosq)
    outs = _gate(xg1, sqs2, cnt, tm1, tm1_W, tm1_b, tmn1_g, tmn1_b,
                 wout=Wout, bout=bout)
    y_head, y_items = outs[-2], outs[-1]

    user_embedding = _user_gather(y_head, user_idx)
    return (user_embedding, y_items)
